# baseline (speedup 1.0000x reference)
import functools

import jax
import jax.numpy as jnp
from jax import lax
from jax.experimental import pallas as pl
from jax.experimental.pallas import tpu as pltpu
from jax.experimental.pallas import tpu_sc as plsc

K_CHEB = 3
_EPS = 1e-5

_NC, _NS = 2, 16
_NW = _NC * _NS


def _pick_chunk(rpw, F):
    def best_le(limit):
        best = 0
        c = 8
        while c <= limit:
            if rpw % c == 0 and 32 * c * F <= 100_000:
                best = c
            c += 8
        return best

    c = best_le(rpw // 2)
    if c == 0:
        c = best_le(rpw)
    return c if c else 8


def _splat(wv, k):
    return lax.gather(
        wv, jnp.full((16, 1), k, jnp.int32),
        lax.GatherDimensionNumbers(
            offset_dims=(), collapsed_slice_dims=(0,),
            start_index_map=(0,)),
        (1,),
        mode=lax.GatherScatterMode.PROMISE_IN_BOUNDS)


@functools.lru_cache(maxsize=None)
def _make_spmm(V, F, with_add):
    nw = min(_NW, V // 8)
    rpw = V // nw
    C = _pick_chunk(rpw, F)
    nj = rpw // C
    NF = F // 16
    nbuf = 2 if nj >= 2 else 1

    mesh = plsc.VectorSubcoreMesh(core_axis_name="c", subcore_axis_name="s")

    buf_types = []
    for _ in range(nbuf):
        buf_types += [
            pltpu.VMEM((C, 16), jnp.float32),
            pltpu.VMEM((8 * C,), jnp.int32),
            pltpu.VMEM((8 * C, F), jnp.float32),
            pltpu.VMEM((C, F), jnp.float32),
            pltpu.VMEM((C, F), jnp.float32),
            pltpu.VMEM((C, F), jnp.float32),
            pltpu.SemaphoreType.DMA,
            pltpu.SemaphoreType.DMA,
            pltpu.SemaphoreType.DMA,
        ]

    def spmm_kernel(*args):
            if with_add:
                x_hbm, cols_hbm, vals_hbm, add_hbm, y_hbm = args[:5]
                scratch = args[5:]
            else:
                x_hbm, cols_hbm, vals_hbm, y_hbm = args[:4]
                add_hbm = None
                scratch = args[4:]
            bufs = [scratch[9 * i:9 * i + 9] for i in range(nbuf)]
            wid = lax.axis_index("s") * _NC + lax.axis_index("c")

            def stage1(buf, j):
                vals_v, idx_v, _, xs_v, _, add_v, sem_in, _, _ = buf
                base = wid * rpw + j * C
                pltpu.async_copy(vals_hbm.at[pl.ds(base, C), :], vals_v,
                                 sem_in)
                pltpu.async_copy(cols_hbm.at[pl.ds(8 * base, 8 * C)], idx_v,
                                 sem_in)
                pltpu.async_copy(x_hbm.at[pl.ds(base, C), :], xs_v, sem_in)
                if with_add:
                    pltpu.async_copy(add_hbm.at[pl.ds(base, C), :], add_v,
                                     sem_in)

            def wait_stage1(buf, j):
                vals_v, idx_v, _, xs_v, _, add_v, sem_in, _, _ = buf
                base = wid * rpw + j * C
                pltpu.make_async_copy(vals_hbm.at[pl.ds(base, C), :], vals_v,
                                      sem_in).wait()
                pltpu.make_async_copy(cols_hbm.at[pl.ds(8 * base, 8 * C)],
                                      idx_v, sem_in).wait()
                pltpu.make_async_copy(x_hbm.at[pl.ds(base, C), :], xs_v,
                                      sem_in).wait()
                if with_add:
                    pltpu.make_async_copy(add_hbm.at[pl.ds(base, C), :],
                                          add_v, sem_in).wait()

            def gather(buf):
                _, idx_v, rows_v, _, _, _, _, sem_g, _ = buf
                pltpu.async_copy(x_hbm.at[idx_v], rows_v, sem_g)

            def wait_gather(buf):
                _, idx_v, rows_v, _, _, _, _, sem_g, _ = buf
                pltpu.make_async_copy(x_hbm.at[idx_v], rows_v, sem_g).wait()

            def put_y(buf, j):
                y_v, sem_y = buf[4], buf[8]
                base = wid * rpw + j * C
                pltpu.async_copy(y_v, y_hbm.at[pl.ds(base, C), :], sem_y)

            def wait_y(buf, j):
                y_v, sem_y = buf[4], buf[8]
                base = wid * rpw + j * C
                pltpu.make_async_copy(y_v, y_hbm.at[pl.ds(base, C), :],
                                      sem_y).wait()

            def compute(buf):
                vals_v, _, rows_v, xs_v, y_v, add_v, _, _, _ = buf

                def row(r, carry):
                    wv = vals_v[r, :]
                    ws = [_splat(wv, k) for k in range(8)]
                    dw = _splat(wv, 8)
                    for f in range(NF):
                        sl = pl.ds(16 * f, 16)
                        acc = dw * xs_v[r, sl]
                        if with_add:
                            acc = acc + add_v[r, sl]
                        for k in range(8):
                            acc = acc + ws[k] * rows_v[8 * r + k, sl]
                        y_v[r, sl] = acc
                    return carry

                lax.fori_loop(0, C, row, 0)

            @pl.when(wid < nw)
            def _work():
                if nbuf == 1:
                    buf = bufs[0]

                    def chunk(j, carry):
                        stage1(buf, j)
                        wait_stage1(buf, j)
                        gather(buf)
                        wait_gather(buf)
                        compute(buf)
                        base = wid * rpw + j * C
                        pltpu.sync_copy(buf[4], y_hbm.at[pl.ds(base, C), :])
                        return carry

                    lax.fori_loop(0, nj, chunk, 0)
                else:
                    b0, b1 = bufs
                    stage1(b0, 0)
                    stage1(b1, 1)
                    wait_stage1(b0, 0)
                    gather(b0)

                    def pair(t, carry):
                        j0 = 2 * t
                        @pl.when(t > 0)
                        def _():
                            wait_y(b0, j0 - 2)

                        @pl.when(j0 + 1 < nj)
                        def _():
                            wait_stage1(b1, j0 + 1)
                            gather(b1)
                        wait_gather(b0)
                        compute(b0)
                        put_y(b0, j0)

                        @pl.when(j0 + 2 < nj)
                        def _():
                            stage1(b0, j0 + 2)

                        @pl.when(j0 + 1 < nj)
                        def _():
                            @pl.when(t > 0)
                            def _():
                                wait_y(b1, j0 - 1)

                            @pl.when(j0 + 2 < nj)
                            def _():
                                wait_stage1(b0, j0 + 2)
                                gather(b0)
                            wait_gather(b1)
                            compute(b1)
                            put_y(b1, j0 + 1)

                            @pl.when(j0 + 3 < nj)
                            def _():
                                stage1(b1, j0 + 3)
                        return carry

                    lax.fori_loop(0, (nj + 1) // 2, pair, 0)
                    wait_y(b0, 0)
                    wait_y(b1, 0)

    return pl.kernel(
        spmm_kernel,
        out_type=jax.ShapeDtypeStruct((V, F), jnp.float32),
        mesh=mesh,
        scratch_types=buf_types,
        compiler_params=pltpu.CompilerParams(use_tc_tiling_on_sc=False),
    )


@functools.lru_cache(maxsize=None)
def _make_spmm2(V, F, with_add):
    nw = min(16, V // 8)
    rpw = V // nw
    C = rpw
    c = 1
    while c <= rpw:
        if rpw % c == 0 and 32 * c * F <= 100_000:
            C = c
        c += 1
    nj = rpw // C
    NF = F // 16

    mesh = plsc.VectorSubcoreMesh(core_axis_name="c", subcore_axis_name="s",
                                  num_cores=1)

    scratch = [
        pltpu.VMEM((C, 16), jnp.float32),
        pltpu.VMEM((8 * C,), jnp.int32),
        pltpu.VMEM((8 * C, F), jnp.float32),
        pltpu.VMEM((C, F), jnp.float32),
        pltpu.VMEM((C, F), jnp.float32),
        pltpu.VMEM((C, F), jnp.float32),
        pltpu.SemaphoreType.DMA,
    ]

    def spmm2_kernel(*args):
        if with_add:
            x_hbm, cols_hbm, vals_hbm, a2_hbm, a3_hbm, o1_hbm, o2_hbm = \
                args[:7]
            rest = args[7:]
        else:
            x_hbm, cols_hbm, vals_hbm, o1_hbm, o2_hbm = args[:5]
            a2_hbm = a3_hbm = None
            rest = args[5:]
        vals_v, idx_v, rows_v, xs_v, y_v, add_v, sem = rest
        wid = lax.axis_index("s")

        def phase(src_hbm, dst_hbm, add_hbm):
            def chunk(j, carry):
                base = wid * rpw + j * C
                pltpu.sync_copy(vals_hbm.at[pl.ds(base, C), :], vals_v)
                pltpu.sync_copy(cols_hbm.at[pl.ds(8 * base, 8 * C)], idx_v)
                pltpu.sync_copy(src_hbm.at[pl.ds(base, C), :], xs_v)
                if add_hbm is not None:
                    pltpu.sync_copy(add_hbm.at[pl.ds(base, C), :], add_v)
                pltpu.async_copy(src_hbm.at[idx_v], rows_v, sem).wait()

                def row(r, carry2):
                    wv = vals_v[r, :]
                    ws = [_splat(wv, k) for k in range(8)]
                    dw = _splat(wv, 8)
                    for f in range(NF):
                        sl = pl.ds(16 * f, 16)
                        acc = dw * xs_v[r, sl]
                        if add_hbm is not None:
                            acc = acc + add_v[r, sl]
                        for k in range(8):
                            acc = acc + ws[k] * rows_v[8 * r + k, sl]
                        y_v[r, sl] = acc
                    return carry2

                lax.fori_loop(0, C, row, 0)
                pltpu.sync_copy(y_v, dst_hbm.at[pl.ds(base, C), :])
                return carry

            lax.fori_loop(0, nj, chunk, 0)

        @pl.when(wid < nw)
        def _p1():
            phase(x_hbm, o1_hbm, a2_hbm)

        plsc.subcore_barrier()

        @pl.when(wid < nw)
        def _p2():
            phase(o1_hbm, o2_hbm, a3_hbm)

    out = jax.ShapeDtypeStruct((V, F), jnp.float32)
    return pl.kernel(
        spmm2_kernel,
        out_type=(out, out),
        mesh=mesh,
        scratch_types=scratch,
        compiler_params=pltpu.CompilerParams(use_tc_tiling_on_sc=False),
    )


_SMALL_V = 768


def _spmm_pair(lap, x):
    V, F = x.shape
    cols8, vals16 = _lap_prep(lap, V)
    return _make_spmm2(V, F, False)(x, cols8, vals16)


def _spmm_pair_add(lap, q, a2, a3):
    V, F = q.shape
    cols8, vals16 = _lap_prep(lap, V)
    return _make_spmm2(V, F, True)(q, cols8, vals16, a2, a3)


def _lap_prep(lap, V):
    _, cols, vals = lap
    e = 8 * V
    vals16 = jnp.concatenate(
        [vals[:e].reshape(V, 8), vals[e:, None],
         jnp.zeros((V, 7), jnp.float32)], axis=1)
    return cols[:e], vals16


def _spmm(lap, x, addend=None):
    V, F = x.shape
    cols8, vals16 = _lap_prep(lap, V)
    if addend is None:
        return _make_spmm(V, F, False)(x, cols8, vals16)
    return _make_spmm(V, F, True)(x, cols8, vals16, addend)


def _row_block(V):
    return 256 if V % 256 == 0 else V


def _dot(a, b):
    return jnp.dot(a, b, preferred_element_type=jnp.float32,
                   precision=lax.Precision.HIGHEST)


def _mm_body(x0_ref, g1_ref, g2_ref, w_ref, b_ref, y_ref, st_ref,
             ssum_ref, ssq_ref, *, Fin, nblocks):
    i = pl.program_id(0)
    y = (_dot(x0_ref[...], w_ref[:Fin, :])
         + _dot(g1_ref[...], w_ref[Fin:2 * Fin, :])
         + _dot(g2_ref[...], w_ref[2 * Fin:, :])
         + b_ref[...])
    y_ref[...] = y

    @pl.when(i == 0)
    def _init():
        ssum_ref[...] = jnp.zeros_like(ssum_ref)
        ssq_ref[...] = jnp.zeros_like(ssq_ref)

    ssum_ref[...] += jnp.sum(y, axis=0, keepdims=True)
    ssq_ref[...] += jnp.sum(y * y, axis=0, keepdims=True)

    @pl.when(i == nblocks - 1)
    def _fin():
        st_ref[...] = jnp.concatenate([ssum_ref[...], ssq_ref[...]], axis=0)


def _mm_plain_body(x0_ref, g1_ref, g2_ref, w_ref, b_ref, y_ref, *, Fin):
    y_ref[...] = (
        _dot(x0_ref[...], w_ref[:Fin, :])
        + _dot(g1_ref[...], w_ref[Fin:2 * Fin, :])
        + _dot(g2_ref[...], w_ref[2 * Fin:, :])
        + b_ref[...])


def _mm3_x_body(x_ref, w_ref, b_ref, q_ref, a2_ref, a3_ref, *, Fout):
    P = _dot(x_ref[...], w_ref[...]) + b_ref[...]
    q_ref[...] = P[:, :Fout]
    a2_ref[...] = P[:, Fout:2 * Fout]
    a3_ref[...] = P[:, 2 * Fout:]


def _mm3_x(x, wcat, bias3):
    V, Fin = x.shape
    Fout = wcat.shape[1] // 3
    BV = _row_block(V)
    nblocks = V // BV
    out_spec = pl.BlockSpec((BV, Fout), lambda i: (i, 0))
    out_shape = jax.ShapeDtypeStruct((V, Fout), jnp.float32)
    return pl.pallas_call(
        functools.partial(_mm3_x_body, Fout=Fout),
        grid=(nblocks,),
        in_specs=[pl.BlockSpec((BV, Fin), lambda i: (i, 0)),
                  pl.BlockSpec((Fin, 3 * Fout), lambda i: (0, 0)),
                  pl.BlockSpec((1, 3 * Fout), lambda i: (0, 0))],
        out_specs=[out_spec, out_spec, out_spec],
        out_shape=[out_shape, out_shape, out_shape],
    )(x, wcat, bias3)


def _mm3_up_body(zc_ref, s_ref, wu_ref, ws_ref, b_ref,
                 q_ref, a2_ref, a3_ref, *, Fout):
    bv4 = zc_ref.shape[0]
    pu = _dot(zc_ref[...], wu_ref[...])
    pu4 = jnp.broadcast_to(pu[:, None, :], (bv4, 4, 3 * Fout))
    pu4 = pu4.reshape(4 * bv4, 3 * Fout)
    P = pu4 + _dot(s_ref[...], ws_ref[...]) + b_ref[...]
    q_ref[...] = P[:, :Fout]
    a2_ref[...] = P[:, Fout:2 * Fout]
    a3_ref[...] = P[:, 2 * Fout:]


def _mm3_up(zc, skip, wu, ws, bias3):
    Vc, Fu = zc.shape
    V, Fs = skip.shape
    Fout = wu.shape[1] // 3
    BV = _row_block(V)
    nblocks = V // BV
    out_spec = pl.BlockSpec((BV, Fout), lambda i: (i, 0))
    out_shape = jax.ShapeDtypeStruct((V, Fout), jnp.float32)
    return pl.pallas_call(
        functools.partial(_mm3_up_body, Fout=Fout),
        grid=(nblocks,),
        in_specs=[pl.BlockSpec((BV // 4, Fu), lambda i: (i, 0)),
                  pl.BlockSpec((BV, Fs), lambda i: (i, 0)),
                  pl.BlockSpec((Fu, 3 * Fout), lambda i: (0, 0)),
                  pl.BlockSpec((Fs, 3 * Fout), lambda i: (0, 0)),
                  pl.BlockSpec((1, 3 * Fout), lambda i: (0, 0))],
        out_specs=[out_spec, out_spec, out_spec],
        out_shape=[out_shape, out_shape, out_shape],
    )(zc, skip, wu, ws, bias3)


def _stats_bn_body(y_ref, gb_ref, z_ref, ssum_ref, ssq_ref, *, V, nblocks):
    i = pl.program_id(0)

    @pl.when(i == 0)
    def _init():
        ssum_ref[...] = jnp.zeros_like(ssum_ref)
        ssq_ref[...] = jnp.zeros_like(ssq_ref)

    y = y_ref[...]

    @pl.when(i < nblocks)
    def _acc():
        ssum_ref[...] += jnp.sum(y, axis=0, keepdims=True)
        ssq_ref[...] += jnp.sum(y * y, axis=0, keepdims=True)
        z_ref[...] = y

    @pl.when(i >= nblocks)
    def _apply():
        m = ssum_ref[...] / V
        var = ssq_ref[...] / V - m * m
        inv = lax.rsqrt(var + _EPS)
        z_ref[...] = jnp.maximum((y - m) * (inv * gb_ref[0:1, :])
                                 + gb_ref[1:2, :], 0.0)


def _stats_bn(y, g, be):
    V, F = y.shape
    BV = _row_block(V)
    nblocks = V // BV
    gb = jnp.stack([g, be], axis=0)
    return pl.pallas_call(
        functools.partial(_stats_bn_body, V=V, nblocks=nblocks),
        grid=(2 * nblocks,),
        in_specs=[pl.BlockSpec((BV, F), lambda i, n=nblocks: (i % n, 0)),
                  pl.BlockSpec((2, F), lambda i: (0, 0))],
        out_specs=pl.BlockSpec((BV, F), lambda i, n=nblocks: (i % n, 0)),
        out_shape=jax.ShapeDtypeStruct((V, F), jnp.float32),
        scratch_shapes=[pltpu.VMEM((1, F), jnp.float32),
                        pltpu.VMEM((1, F), jnp.float32)],
    )(y, gb)


def _mm_add_body(x_ref, w_ref, a_ref, b_ref, y_ref, st_ref,
                 ssum_ref, ssq_ref, *, nblocks, has_addend, with_stats):
    i = pl.program_id(0)
    y = _dot(x_ref[...], w_ref[...])
    if has_addend:
        y = y + a_ref[...]
    if b_ref is not None:
        y = y + b_ref[...]
    y_ref[...] = y
    if with_stats:
        @pl.when(i == 0)
        def _init():
            ssum_ref[...] = jnp.zeros_like(ssum_ref)
            ssq_ref[...] = jnp.zeros_like(ssq_ref)

        ssum_ref[...] += jnp.sum(y, axis=0, keepdims=True)
        ssq_ref[...] += jnp.sum(y * y, axis=0, keepdims=True)

        @pl.when(i == nblocks - 1)
        def _fin():
            st_ref[...] = jnp.concatenate([ssum_ref[...], ssq_ref[...]],
                                          axis=0)


def _mm_add(x, w, addend=None, bias=None, with_stats=False):
    V, Fin = x.shape
    Fout = w.shape[1]
    BV = _row_block(V)
    nblocks = V // BV
    row_spec = pl.BlockSpec((BV, Fin), lambda i: (i, 0))
    w_spec = pl.BlockSpec((Fin, Fout), lambda i: (0, 0))
    vec_spec = pl.BlockSpec((1, Fout), lambda i: (0, 0))
    y_spec = pl.BlockSpec((BV, Fout), lambda i: (i, 0))
    a_spec = pl.BlockSpec((BV, Fout), lambda i: (i, 0))
    args = [x, w]
    in_specs = [row_spec, w_spec]
    has_addend = addend is not None
    if has_addend:
        args.append(addend)
        in_specs.append(a_spec)

    if bias is not None:
        args.append(bias.reshape(1, Fout))
        in_specs.append(vec_spec)

    def body(*refs):
        idx = 2
        a_ref = None
        b_ref = None
        if has_addend:
            a_ref = refs[idx]; idx += 1
        if bias is not None:
            b_ref = refs[idx]; idx += 1
        if with_stats:
            y_ref, st_ref = refs[idx], refs[idx + 1]
            ssum_ref, ssq_ref = refs[idx + 2], refs[idx + 3]
        else:
            y_ref, st_ref, ssum_ref, ssq_ref = refs[idx], None, None, None
        _mm_add_body(refs[0], refs[1], a_ref, b_ref, y_ref, st_ref,
                     ssum_ref, ssq_ref, nblocks=nblocks,
                     has_addend=has_addend, with_stats=with_stats)

    if with_stats:
        st_spec = pl.BlockSpec((2, Fout), lambda i: (0, 0))
        y, st = pl.pallas_call(
            body,
            grid=(nblocks,),
            in_specs=in_specs,
            out_specs=[y_spec, st_spec],
            out_shape=[jax.ShapeDtypeStruct((V, Fout), jnp.float32),
                       jax.ShapeDtypeStruct((2, Fout), jnp.float32)],
            scratch_shapes=[pltpu.VMEM((1, Fout), jnp.float32),
                            pltpu.VMEM((1, Fout), jnp.float32)],
        )(*args)
        return y, st
    y = pl.pallas_call(
        body,
        grid=(nblocks,),
        in_specs=in_specs,
        out_specs=y_spec,
        out_shape=jax.ShapeDtypeStruct((V, Fout), jnp.float32),
    )(*args)
    return y, None


def _cheb_combine(x0, g1, g2, w_stack, b, with_stats):
    V, Fin = x0.shape
    Fout = w_stack.shape[1]
    BV = _row_block(V)
    nblocks = V // BV
    b2 = b.reshape(1, Fout)
    row_spec = pl.BlockSpec((BV, Fin), lambda i: (i, 0))
    w_spec = pl.BlockSpec((3 * Fin, Fout), lambda i: (0, 0))
    b_spec = pl.BlockSpec((1, Fout), lambda i: (0, 0))
    y_spec = pl.BlockSpec((BV, Fout), lambda i: (i, 0))
    if with_stats:
        st_spec = pl.BlockSpec((2, Fout), lambda i: (0, 0))
        y, st = pl.pallas_call(
            functools.partial(_mm_body, Fin=Fin, nblocks=nblocks),
            grid=(nblocks,),
            in_specs=[row_spec, row_spec, row_spec, w_spec, b_spec],
            out_specs=[y_spec, st_spec],
            out_shape=[jax.ShapeDtypeStruct((V, Fout), jnp.float32),
                       jax.ShapeDtypeStruct((2, Fout), jnp.float32)],
            scratch_shapes=[pltpu.VMEM((1, Fout), jnp.float32),
                            pltpu.VMEM((1, Fout), jnp.float32)],
        )(x0, g1, g2, w_stack, b2)
        return y, st
    y = pl.pallas_call(
        functools.partial(_mm_plain_body, Fin=Fin),
        grid=(nblocks,),
        in_specs=[row_spec, row_spec, row_spec, w_spec, b_spec],
        out_specs=y_spec,
        out_shape=jax.ShapeDtypeStruct((V, Fout), jnp.float32),
    )(x0, g1, g2, w_stack, b2)
    return y, None


def _combine_bn_body(x0_ref, g1_ref, g2_ref, w_ref, b_ref, gb_ref,
                     z_ref, p_ref, y_sc, ssum_ref, ssq_ref,
                     *, Fin, nblocks, V, BV, pool):
    i = pl.program_id(0)

    @pl.when(i == 0)
    def _init():
        ssum_ref[...] = jnp.zeros_like(ssum_ref)
        ssq_ref[...] = jnp.zeros_like(ssq_ref)

    @pl.when(i < nblocks)
    def _mm():
        y = (_dot(x0_ref[...], w_ref[:Fin, :])
             + _dot(g1_ref[...], w_ref[Fin:2 * Fin, :])
             + _dot(g2_ref[...], w_ref[2 * Fin:, :])
             + b_ref[...])
        y_sc[pl.ds(i * BV, BV), :] = y
        ssum_ref[...] += jnp.sum(y, axis=0, keepdims=True)
        ssq_ref[...] += jnp.sum(y * y, axis=0, keepdims=True)

    @pl.when(i >= nblocks)
    def _bn():
        y = y_sc[pl.ds((i - nblocks) * BV, BV), :]
        m = ssum_ref[...] / V
        var = ssq_ref[...] / V - m * m
        inv = lax.rsqrt(var + _EPS)
        z = jnp.maximum((y - m) * (inv * gb_ref[0:1, :]) + gb_ref[1:2, :],
                        0.0)
        z_ref[...] = z
        if pool:
            f = z.shape[1]
            p_ref[...] = jnp.mean(z.reshape(BV // 4, 4, f), axis=1)


def _combine_bn(x0, g1, g2, w_stack, b, g, be, pool):
    V, Fin = x0.shape
    Fout = w_stack.shape[1]
    BV = _row_block(V)
    n = V // BV
    gb = jnp.stack([g, be], axis=0)

    def row_map(i, nb=n):
        return (jnp.where(i < nb, i, 0), 0)

    def out_map(i, nb=n):
        return (jnp.where(i < nb, 0, i - nb), 0)

    in_specs = [pl.BlockSpec((BV, Fin), row_map),
                pl.BlockSpec((BV, Fin), row_map),
                pl.BlockSpec((BV, Fin), row_map),
                pl.BlockSpec((3 * Fin, Fout), lambda i: (0, 0)),
                pl.BlockSpec((1, Fout), lambda i: (0, 0)),
                pl.BlockSpec((2, Fout), lambda i: (0, 0))]
    out_specs = [pl.BlockSpec((BV, Fout), out_map)]
    out_shape = [jax.ShapeDtypeStruct((V, Fout), jnp.float32)]
    if pool:
        out_specs.append(pl.BlockSpec((BV // 4, Fout), out_map))
        out_shape.append(jax.ShapeDtypeStruct((V // 4, Fout), jnp.float32))
    def body(*refs):
        if pool:
            (x0_r, g1_r, g2_r, w_r, b_r, gb_r, z_r, p_r, y_sc, s1, s2) = refs
        else:
            (x0_r, g1_r, g2_r, w_r, b_r, gb_r, z_r, y_sc, s1, s2) = refs
            p_r = None
        _combine_bn_body(x0_r, g1_r, g2_r, w_r, b_r, gb_r, z_r, p_r,
                         y_sc, s1, s2, Fin=Fin, nblocks=n, V=V, BV=BV,
                         pool=pool)

    res = pl.pallas_call(
        body,
        grid=(2 * n,),
        in_specs=in_specs,
        out_specs=out_specs if pool else out_specs[0],
        out_shape=out_shape if pool else out_shape[0],
        scratch_shapes=[pltpu.VMEM((V, Fout), jnp.float32),
                        pltpu.VMEM((1, Fout), jnp.float32),
                        pltpu.VMEM((1, Fout), jnp.float32)],
    )(x0, g1, g2, w_stack, b.reshape(1, Fout), gb)
    return res


def _bn_body(y_ref, st_ref, gb_ref, z_ref, *, V):
    m = st_ref[0:1, :] / V
    var = st_ref[1:2, :] / V - m * m
    inv = lax.rsqrt(var + _EPS)
    z_ref[...] = jnp.maximum(
        (y_ref[...] - m) * (inv * gb_ref[0:1, :]) + gb_ref[1:2, :], 0.0)


def _bn_pool_body(y_ref, st_ref, gb_ref, z_ref, p_ref, *, V):
    m = st_ref[0:1, :] / V
    var = st_ref[1:2, :] / V - m * m
    inv = lax.rsqrt(var + _EPS)
    z = jnp.maximum(
        (y_ref[...] - m) * (inv * gb_ref[0:1, :]) + gb_ref[1:2, :], 0.0)
    z_ref[...] = z
    bv, f = z.shape
    p_ref[...] = jnp.mean(z.reshape(bv // 4, 4, f), axis=1)


def _bn_relu(y, st, g, be, pool=False):
    V, F = y.shape
    BV = _row_block(V)
    gb = jnp.stack([g, be], axis=0)
    in_specs = [pl.BlockSpec((BV, F), lambda i: (i, 0)),
                pl.BlockSpec((2, F), lambda i: (0, 0)),
                pl.BlockSpec((2, F), lambda i: (0, 0))]
    if not pool:
        return pl.pallas_call(
            functools.partial(_bn_body, V=V),
            grid=(V // BV,),
            in_specs=in_specs,
            out_specs=pl.BlockSpec((BV, F), lambda i: (i, 0)),
            out_shape=jax.ShapeDtypeStruct((V, F), jnp.float32),
        )(y, st, gb)
    return pl.pallas_call(
        functools.partial(_bn_pool_body, V=V),
        grid=(V // BV,),
        in_specs=in_specs,
        out_specs=[pl.BlockSpec((BV, F), lambda i: (i, 0)),
                   pl.BlockSpec((BV // 4, F), lambda i: (i, 0))],
        out_shape=[jax.ShapeDtypeStruct((V, F), jnp.float32),
                   jax.ShapeDtypeStruct((V // 4, F), jnp.float32)],
    )(y, st, gb)


def _pool_body(x_ref, p_ref):
    p_ref[...] = jnp.mean(x_ref[...], axis=1)


def _pool(x):
    V, F = x.shape
    Vp = V // 4
    BP = _row_block(Vp)
    x3 = x.reshape(Vp, 4, F)
    return pl.pallas_call(
        _pool_body,
        grid=(Vp // BP,),
        in_specs=[pl.BlockSpec((BP, 4, F), lambda i: (i, 0, 0))],
        out_specs=pl.BlockSpec((BP, F), lambda i: (i, 0)),
        out_shape=jax.ShapeDtypeStruct((Vp, F), jnp.float32),
    )(x3)


def _unpool_body(x_ref, u_ref):
    b, _, f = u_ref.shape
    u_ref[...] = jnp.broadcast_to(x_ref[...][:, None, :], (b, 4, f))


def _unpool(x):
    Vc, F = x.shape
    BP = _row_block(Vc)
    u = pl.pallas_call(
        _unpool_body,
        grid=(Vc // BP,),
        in_specs=[pl.BlockSpec((BP, F), lambda i: (i, 0))],
        out_specs=pl.BlockSpec((BP, 4, F), lambda i: (i, 0, 0)),
        out_shape=jax.ShapeDtypeStruct((Vc, 4, F), jnp.float32),
    )(x)
    return u.reshape(4 * Vc, F)


def _prep_weights(w):
    K, Fin, Fout = w.shape
    weff = jnp.transpose(w.reshape(Fin, K, Fout), (1, 0, 2))
    return weff[0] - weff[2], weff[1], 2.0 * weff[2]


def _spmm_chain(lap, x0):
    if x0.shape[0] <= _SMALL_V:
        return _spmm_pair(lap, x0)
    g1 = _spmm(lap, x0)
    return g1, _spmm(lap, g1)


def _cheb_fin(lap, x0, p, name, with_stats):
    wa, wb, wc = _prep_weights(p[name + '_w'])
    w_stack = jnp.concatenate([wa, wb, wc], axis=0)
    g1, g2 = _spmm_chain(lap, x0)
    return _cheb_combine(x0, g1, g2, w_stack, p[name + '_b'], with_stats)


def _fout_weights(p, name):
    wa, wb, wc = _prep_weights(p[name + '_w'])
    b = p[name + '_b']
    Fout = wa.shape[1]
    wcat = jnp.concatenate([wc, wb, wa], axis=1)
    bias3 = jnp.concatenate(
        [jnp.zeros((2 * Fout,), jnp.float32), b]).reshape(1, 3 * Fout)
    return wcat, bias3


def _cheb_fout_x(lap, x0, p, name):
    wcat, bias3 = _fout_weights(p, name)
    q, a2, a3 = _mm3_x(x0, wcat, bias3)
    if q.shape[0] <= _SMALL_V:
        return _spmm_pair_add(lap, q, a2, a3)[1]
    t = _spmm(lap, q, addend=a2)
    return _spmm(lap, t, addend=a3)


def _cheb_fout_up(lap, zc, skip, p, name):
    wcat, bias3 = _fout_weights(p, name)
    Fu = zc.shape[1]
    q, a2, a3 = _mm3_up(zc, skip, wcat[:Fu], wcat[Fu:], bias3)
    if q.shape[0] <= _SMALL_V:
        return _spmm_pair_add(lap, q, a2, a3)[1]
    t = _spmm(lap, q, addend=a2)
    return _spmm(lap, t, addend=a3)


def _cbn_fin(lap, x0, p, name, pool=False):
    wa, wb, wc = _prep_weights(p[name + '_w'])
    w_stack = jnp.concatenate([wa, wb, wc], axis=0)
    g1, g2 = _spmm_chain(lap, x0)
    return _combine_bn(x0, g1, g2, w_stack, p[name + '_b'],
                       p[name + '_g'], p[name + '_be'], pool)


def kernel(x, params, laps):
    p = params
    x0 = x[0]

    x5a = _cbn_fin(laps[5], x0, p, 'e5a')
    x5, x5p = _cbn_fin(laps[5], x5a, p, 'e5b', pool=True)
    x4, x4p = _cbn_fin(laps[4], x5p, p, 'e4', pool=True)
    x3, x3p = _cbn_fin(laps[3], x4p, p, 'e3', pool=True)
    x2, x2p = _cbn_fin(laps[2], x3p, p, 'e2', pool=True)
    _, x1p = _cbn_fin(laps[1], x2p, p, 'e1', pool=True)
    xb = _cheb_fin(laps[0], x1p, p, 'e0', False)[0]

    z1 = _cbn_fin(laps[1], _unpool(xb), p, 'd1a')
    d1 = _cheb_fout_x(laps[1], z1, p, 'd1b')
    d2 = _stats_bn(_cheb_fout_up(laps[2], d1, x2, p, 'd2'),
                   p['d2_g'], p['d2_be'])
    d3 = _stats_bn(_cheb_fout_up(laps[3], d2, x3, p, 'd3'),
                   p['d3_g'], p['d3_be'])
    d4 = _stats_bn(_cheb_fout_up(laps[4], d3, x4, p, 'd4'),
                   p['d4_g'], p['d4_be'])
    d5 = _stats_bn(_cheb_fout_up(laps[5], d4, x5, p, 'd5'),
                   p['d5_g'], p['d5_be'])
    out = _cheb_fin(laps[5], d5, p, 'df', False)[0]
    return out.reshape(1, out.shape[0], 1)


# speedup vs baseline: 1.0163x; 1.0163x over previous
import functools

import jax
import jax.numpy as jnp
from jax import lax
from jax.experimental import pallas as pl
from jax.experimental.pallas import tpu as pltpu
from jax.experimental.pallas import tpu_sc as plsc

K_CHEB = 3
_EPS = 1e-5

_NC, _NS = 2, 16
_NW = _NC * _NS


def _pick_chunk(rpw, F):
    def best_le(limit):
        best = 0
        c = 8
        while c <= limit:
            if rpw % c == 0 and 32 * c * F <= 100_000:
                best = c
            c += 8
        return best

    c = best_le(rpw // 2)
    if c == 0:
        c = best_le(rpw)
    return c if c else 8


def _splat(wv, k):
    return lax.gather(
        wv, jnp.full((16, 1), k, jnp.int32),
        lax.GatherDimensionNumbers(
            offset_dims=(), collapsed_slice_dims=(0,),
            start_index_map=(0,)),
        (1,),
        mode=lax.GatherScatterMode.PROMISE_IN_BOUNDS)


@functools.lru_cache(maxsize=None)
def _make_spmm(V, F, with_add):
    nw = min(_NW, V // 8)
    rpw = V // nw
    C = _pick_chunk(rpw, F)
    nj = rpw // C
    NF = F // 16
    nbuf = 2 if nj >= 2 else 1

    mesh = plsc.VectorSubcoreMesh(core_axis_name="c", subcore_axis_name="s")

    buf_types = []
    for _ in range(nbuf):
        buf_types += [
            pltpu.VMEM((C, 16), jnp.float32),
            pltpu.VMEM((8 * C,), jnp.int32),
            pltpu.VMEM((8 * C, F), jnp.float32),
            pltpu.VMEM((C, F), jnp.float32),
            pltpu.VMEM((C, F), jnp.float32),
            pltpu.VMEM((C, F), jnp.float32),
            pltpu.SemaphoreType.DMA,
            pltpu.SemaphoreType.DMA,
            pltpu.SemaphoreType.DMA,
        ]

    def spmm_kernel(*args):
            if with_add:
                x_hbm, cols_hbm, vals_hbm, add_hbm, y_hbm = args[:5]
                scratch = args[5:]
            else:
                x_hbm, cols_hbm, vals_hbm, y_hbm = args[:4]
                add_hbm = None
                scratch = args[4:]
            bufs = [scratch[9 * i:9 * i + 9] for i in range(nbuf)]
            wid = lax.axis_index("s") * _NC + lax.axis_index("c")

            def stage1(buf, j):
                vals_v, idx_v, _, xs_v, _, add_v, sem_in, _, _ = buf
                base = wid * rpw + j * C
                pltpu.async_copy(vals_hbm.at[pl.ds(base, C), :], vals_v,
                                 sem_in)
                pltpu.async_copy(cols_hbm.at[pl.ds(8 * base, 8 * C)], idx_v,
                                 sem_in)
                pltpu.async_copy(x_hbm.at[pl.ds(base, C), :], xs_v, sem_in)
                if with_add:
                    pltpu.async_copy(add_hbm.at[pl.ds(base, C), :], add_v,
                                     sem_in)

            def wait_stage1(buf, j):
                vals_v, idx_v, _, xs_v, _, add_v, sem_in, _, _ = buf
                base = wid * rpw + j * C
                pltpu.make_async_copy(vals_hbm.at[pl.ds(base, C), :], vals_v,
                                      sem_in).wait()
                pltpu.make_async_copy(cols_hbm.at[pl.ds(8 * base, 8 * C)],
                                      idx_v, sem_in).wait()
                pltpu.make_async_copy(x_hbm.at[pl.ds(base, C), :], xs_v,
                                      sem_in).wait()
                if with_add:
                    pltpu.make_async_copy(add_hbm.at[pl.ds(base, C), :],
                                          add_v, sem_in).wait()

            def gather(buf):
                _, idx_v, rows_v, _, _, _, _, sem_g, _ = buf
                pltpu.async_copy(x_hbm.at[idx_v], rows_v, sem_g)

            def wait_gather(buf):
                _, idx_v, rows_v, _, _, _, _, sem_g, _ = buf
                pltpu.make_async_copy(x_hbm.at[idx_v], rows_v, sem_g).wait()

            def put_y(buf, j):
                y_v, sem_y = buf[4], buf[8]
                base = wid * rpw + j * C
                pltpu.async_copy(y_v, y_hbm.at[pl.ds(base, C), :], sem_y)

            def wait_y(buf, j):
                y_v, sem_y = buf[4], buf[8]
                base = wid * rpw + j * C
                pltpu.make_async_copy(y_v, y_hbm.at[pl.ds(base, C), :],
                                      sem_y).wait()

            def compute(buf):
                vals_v, _, rows_v, xs_v, y_v, add_v, _, _, _ = buf

                def row(r, carry):
                    wv = vals_v[r, :]
                    ws = [_splat(wv, k) for k in range(8)]
                    dw = _splat(wv, 8)
                    for f in range(NF):
                        sl = pl.ds(16 * f, 16)
                        acc = dw * xs_v[r, sl]
                        if with_add:
                            acc = acc + add_v[r, sl]
                        for k in range(8):
                            acc = acc + ws[k] * rows_v[8 * r + k, sl]
                        y_v[r, sl] = acc
                    return carry

                lax.fori_loop(0, C, row, 0)

            @pl.when(wid < nw)
            def _work():
                if nbuf == 1:
                    buf = bufs[0]

                    def chunk(j, carry):
                        stage1(buf, j)
                        wait_stage1(buf, j)
                        gather(buf)
                        wait_gather(buf)
                        compute(buf)
                        base = wid * rpw + j * C
                        pltpu.sync_copy(buf[4], y_hbm.at[pl.ds(base, C), :])
                        return carry

                    lax.fori_loop(0, nj, chunk, 0)
                else:
                    b0, b1 = bufs
                    stage1(b0, 0)
                    stage1(b1, 1)
                    wait_stage1(b0, 0)
                    gather(b0)

                    def pair(t, carry):
                        j0 = 2 * t
                        @pl.when(t > 0)
                        def _():
                            wait_y(b0, j0 - 2)

                        @pl.when(j0 + 1 < nj)
                        def _():
                            wait_stage1(b1, j0 + 1)
                            gather(b1)
                        wait_gather(b0)
                        compute(b0)
                        put_y(b0, j0)

                        @pl.when(j0 + 2 < nj)
                        def _():
                            stage1(b0, j0 + 2)

                        @pl.when(j0 + 1 < nj)
                        def _():
                            @pl.when(t > 0)
                            def _():
                                wait_y(b1, j0 - 1)

                            @pl.when(j0 + 2 < nj)
                            def _():
                                wait_stage1(b0, j0 + 2)
                                gather(b0)
                            wait_gather(b1)
                            compute(b1)
                            put_y(b1, j0 + 1)

                            @pl.when(j0 + 3 < nj)
                            def _():
                                stage1(b1, j0 + 3)
                        return carry

                    lax.fori_loop(0, (nj + 1) // 2, pair, 0)
                    wait_y(b0, 0)
                    wait_y(b1, 0)

    return pl.kernel(
        spmm_kernel,
        out_type=jax.ShapeDtypeStruct((V, F), jnp.float32),
        mesh=mesh,
        scratch_types=buf_types,
        compiler_params=pltpu.CompilerParams(use_tc_tiling_on_sc=False),
    )


@functools.lru_cache(maxsize=None)
def _make_spmm2(V, F, with_add):
    nw = min(16, V // 8)
    rpw = V // nw
    C = rpw
    c = 1
    while c <= rpw:
        if rpw % c == 0 and 32 * c * F <= 100_000:
            C = c
        c += 1
    nj = rpw // C
    NF = F // 16

    mesh = plsc.VectorSubcoreMesh(core_axis_name="c", subcore_axis_name="s",
                                  num_cores=1)

    scratch = [
        pltpu.VMEM((C, 16), jnp.float32),
        pltpu.VMEM((8 * C,), jnp.int32),
        pltpu.VMEM((8 * C, F), jnp.float32),
        pltpu.VMEM((C, F), jnp.float32),
        pltpu.VMEM((C, F), jnp.float32),
        pltpu.VMEM((C, F), jnp.float32),
        pltpu.SemaphoreType.DMA,
    ]

    def spmm2_kernel(*args):
        if with_add:
            x_hbm, cols_hbm, vals_hbm, a2_hbm, a3_hbm, o1_hbm, o2_hbm = \
                args[:7]
            rest = args[7:]
        else:
            x_hbm, cols_hbm, vals_hbm, o1_hbm, o2_hbm = args[:5]
            a2_hbm = a3_hbm = None
            rest = args[5:]
        vals_v, idx_v, rows_v, xs_v, y_v, add_v, sem = rest
        wid = lax.axis_index("s")

        def phase(src_hbm, dst_hbm, add_hbm):
            def chunk(j, carry):
                base = wid * rpw + j * C
                pltpu.sync_copy(vals_hbm.at[pl.ds(base, C), :], vals_v)
                pltpu.sync_copy(cols_hbm.at[pl.ds(8 * base, 8 * C)], idx_v)
                pltpu.sync_copy(src_hbm.at[pl.ds(base, C), :], xs_v)
                if add_hbm is not None:
                    pltpu.sync_copy(add_hbm.at[pl.ds(base, C), :], add_v)
                pltpu.async_copy(src_hbm.at[idx_v], rows_v, sem).wait()

                def row(r, carry2):
                    wv = vals_v[r, :]
                    ws = [_splat(wv, k) for k in range(8)]
                    dw = _splat(wv, 8)
                    for f in range(NF):
                        sl = pl.ds(16 * f, 16)
                        acc = dw * xs_v[r, sl]
                        if add_hbm is not None:
                            acc = acc + add_v[r, sl]
                        for k in range(8):
                            acc = acc + ws[k] * rows_v[8 * r + k, sl]
                        y_v[r, sl] = acc
                    return carry2

                lax.fori_loop(0, C, row, 0)
                pltpu.sync_copy(y_v, dst_hbm.at[pl.ds(base, C), :])
                return carry

            lax.fori_loop(0, nj, chunk, 0)

        @pl.when(wid < nw)
        def _p1():
            phase(x_hbm, o1_hbm, a2_hbm)

        plsc.subcore_barrier()

        @pl.when(wid < nw)
        def _p2():
            phase(o1_hbm, o2_hbm, a3_hbm)

    out = jax.ShapeDtypeStruct((V, F), jnp.float32)
    return pl.kernel(
        spmm2_kernel,
        out_type=(out, out),
        mesh=mesh,
        scratch_types=scratch,
        compiler_params=pltpu.CompilerParams(use_tc_tiling_on_sc=False),
    )


_SMALL_V = 192


def _spmm_pair(lap, x):
    V, F = x.shape
    cols8, vals16 = _lap_prep(lap, V)
    return _make_spmm2(V, F, False)(x, cols8, vals16)


def _spmm_pair_add(lap, q, a2, a3):
    V, F = q.shape
    cols8, vals16 = _lap_prep(lap, V)
    return _make_spmm2(V, F, True)(q, cols8, vals16, a2, a3)


def _lap_prep(lap, V):
    _, cols, vals = lap
    e = 8 * V
    vals16 = jnp.concatenate(
        [vals[:e].reshape(V, 8), vals[e:, None],
         jnp.zeros((V, 7), jnp.float32)], axis=1)
    return cols[:e], vals16


def _spmm(lap, x, addend=None):
    V, F = x.shape
    cols8, vals16 = _lap_prep(lap, V)
    if addend is None:
        return _make_spmm(V, F, False)(x, cols8, vals16)
    return _make_spmm(V, F, True)(x, cols8, vals16, addend)


def _row_block(V):
    return 256 if V % 256 == 0 else V


def _dot(a, b):
    return jnp.dot(a, b, preferred_element_type=jnp.float32,
                   precision=lax.Precision.HIGHEST)


def _mm_body(x0_ref, g1_ref, g2_ref, w_ref, b_ref, y_ref, st_ref,
             ssum_ref, ssq_ref, *, Fin, nblocks):
    i = pl.program_id(0)
    y = (_dot(x0_ref[...], w_ref[:Fin, :])
         + _dot(g1_ref[...], w_ref[Fin:2 * Fin, :])
         + _dot(g2_ref[...], w_ref[2 * Fin:, :])
         + b_ref[...])
    y_ref[...] = y

    @pl.when(i == 0)
    def _init():
        ssum_ref[...] = jnp.zeros_like(ssum_ref)
        ssq_ref[...] = jnp.zeros_like(ssq_ref)

    ssum_ref[...] += jnp.sum(y, axis=0, keepdims=True)
    ssq_ref[...] += jnp.sum(y * y, axis=0, keepdims=True)

    @pl.when(i == nblocks - 1)
    def _fin():
        st_ref[...] = jnp.concatenate([ssum_ref[...], ssq_ref[...]], axis=0)


def _mm_plain_body(x0_ref, g1_ref, g2_ref, w_ref, b_ref, y_ref, *, Fin):
    y_ref[...] = (
        _dot(x0_ref[...], w_ref[:Fin, :])
        + _dot(g1_ref[...], w_ref[Fin:2 * Fin, :])
        + _dot(g2_ref[...], w_ref[2 * Fin:, :])
        + b_ref[...])


def _mm3_x_body(x_ref, w_ref, b_ref, q_ref, a2_ref, a3_ref, *, Fout):
    P = _dot(x_ref[...], w_ref[...]) + b_ref[...]
    q_ref[...] = P[:, :Fout]
    a2_ref[...] = P[:, Fout:2 * Fout]
    a3_ref[...] = P[:, 2 * Fout:]


def _mm3_x(x, wcat, bias3):
    V, Fin = x.shape
    Fout = wcat.shape[1] // 3
    BV = _row_block(V)
    nblocks = V // BV
    out_spec = pl.BlockSpec((BV, Fout), lambda i: (i, 0))
    out_shape = jax.ShapeDtypeStruct((V, Fout), jnp.float32)
    return pl.pallas_call(
        functools.partial(_mm3_x_body, Fout=Fout),
        grid=(nblocks,),
        in_specs=[pl.BlockSpec((BV, Fin), lambda i: (i, 0)),
                  pl.BlockSpec((Fin, 3 * Fout), lambda i: (0, 0)),
                  pl.BlockSpec((1, 3 * Fout), lambda i: (0, 0))],
        out_specs=[out_spec, out_spec, out_spec],
        out_shape=[out_shape, out_shape, out_shape],
    )(x, wcat, bias3)


def _mm3_up_body(zc_ref, s_ref, wu_ref, ws_ref, b_ref,
                 q_ref, a2_ref, a3_ref, *, Fout):
    bv4 = zc_ref.shape[0]
    pu = _dot(zc_ref[...], wu_ref[...])
    pu4 = jnp.broadcast_to(pu[:, None, :], (bv4, 4, 3 * Fout))
    pu4 = pu4.reshape(4 * bv4, 3 * Fout)
    P = pu4 + _dot(s_ref[...], ws_ref[...]) + b_ref[...]
    q_ref[...] = P[:, :Fout]
    a2_ref[...] = P[:, Fout:2 * Fout]
    a3_ref[...] = P[:, 2 * Fout:]


def _mm3_up(zc, skip, wu, ws, bias3):
    Vc, Fu = zc.shape
    V, Fs = skip.shape
    Fout = wu.shape[1] // 3
    BV = _row_block(V)
    nblocks = V // BV
    out_spec = pl.BlockSpec((BV, Fout), lambda i: (i, 0))
    out_shape = jax.ShapeDtypeStruct((V, Fout), jnp.float32)
    return pl.pallas_call(
        functools.partial(_mm3_up_body, Fout=Fout),
        grid=(nblocks,),
        in_specs=[pl.BlockSpec((BV // 4, Fu), lambda i: (i, 0)),
                  pl.BlockSpec((BV, Fs), lambda i: (i, 0)),
                  pl.BlockSpec((Fu, 3 * Fout), lambda i: (0, 0)),
                  pl.BlockSpec((Fs, 3 * Fout), lambda i: (0, 0)),
                  pl.BlockSpec((1, 3 * Fout), lambda i: (0, 0))],
        out_specs=[out_spec, out_spec, out_spec],
        out_shape=[out_shape, out_shape, out_shape],
    )(zc, skip, wu, ws, bias3)


def _stats_bn_body(y_ref, gb_ref, z_ref, ssum_ref, ssq_ref, *, V, nblocks):
    i = pl.program_id(0)

    @pl.when(i == 0)
    def _init():
        ssum_ref[...] = jnp.zeros_like(ssum_ref)
        ssq_ref[...] = jnp.zeros_like(ssq_ref)

    y = y_ref[...]

    @pl.when(i < nblocks)
    def _acc():
        ssum_ref[...] += jnp.sum(y, axis=0, keepdims=True)
        ssq_ref[...] += jnp.sum(y * y, axis=0, keepdims=True)
        z_ref[...] = y

    @pl.when(i >= nblocks)
    def _apply():
        m = ssum_ref[...] / V
        var = ssq_ref[...] / V - m * m
        inv = lax.rsqrt(var + _EPS)
        z_ref[...] = jnp.maximum((y - m) * (inv * gb_ref[0:1, :])
                                 + gb_ref[1:2, :], 0.0)


def _stats_bn(y, g, be):
    V, F = y.shape
    BV = _row_block(V)
    nblocks = V // BV
    gb = jnp.stack([g, be], axis=0)
    return pl.pallas_call(
        functools.partial(_stats_bn_body, V=V, nblocks=nblocks),
        grid=(2 * nblocks,),
        in_specs=[pl.BlockSpec((BV, F), lambda i, n=nblocks: (i % n, 0)),
                  pl.BlockSpec((2, F), lambda i: (0, 0))],
        out_specs=pl.BlockSpec((BV, F), lambda i, n=nblocks: (i % n, 0)),
        out_shape=jax.ShapeDtypeStruct((V, F), jnp.float32),
        scratch_shapes=[pltpu.VMEM((1, F), jnp.float32),
                        pltpu.VMEM((1, F), jnp.float32)],
    )(y, gb)


def _mm_add_body(x_ref, w_ref, a_ref, b_ref, y_ref, st_ref,
                 ssum_ref, ssq_ref, *, nblocks, has_addend, with_stats):
    i = pl.program_id(0)
    y = _dot(x_ref[...], w_ref[...])
    if has_addend:
        y = y + a_ref[...]
    if b_ref is not None:
        y = y + b_ref[...]
    y_ref[...] = y
    if with_stats:
        @pl.when(i == 0)
        def _init():
            ssum_ref[...] = jnp.zeros_like(ssum_ref)
            ssq_ref[...] = jnp.zeros_like(ssq_ref)

        ssum_ref[...] += jnp.sum(y, axis=0, keepdims=True)
        ssq_ref[...] += jnp.sum(y * y, axis=0, keepdims=True)

        @pl.when(i == nblocks - 1)
        def _fin():
            st_ref[...] = jnp.concatenate([ssum_ref[...], ssq_ref[...]],
                                          axis=0)


def _mm_add(x, w, addend=None, bias=None, with_stats=False):
    V, Fin = x.shape
    Fout = w.shape[1]
    BV = _row_block(V)
    nblocks = V // BV
    row_spec = pl.BlockSpec((BV, Fin), lambda i: (i, 0))
    w_spec = pl.BlockSpec((Fin, Fout), lambda i: (0, 0))
    vec_spec = pl.BlockSpec((1, Fout), lambda i: (0, 0))
    y_spec = pl.BlockSpec((BV, Fout), lambda i: (i, 0))
    a_spec = pl.BlockSpec((BV, Fout), lambda i: (i, 0))
    args = [x, w]
    in_specs = [row_spec, w_spec]
    has_addend = addend is not None
    if has_addend:
        args.append(addend)
        in_specs.append(a_spec)

    if bias is not None:
        args.append(bias.reshape(1, Fout))
        in_specs.append(vec_spec)

    def body(*refs):
        idx = 2
        a_ref = None
        b_ref = None
        if has_addend:
            a_ref = refs[idx]; idx += 1
        if bias is not None:
            b_ref = refs[idx]; idx += 1
        if with_stats:
            y_ref, st_ref = refs[idx], refs[idx + 1]
            ssum_ref, ssq_ref = refs[idx + 2], refs[idx + 3]
        else:
            y_ref, st_ref, ssum_ref, ssq_ref = refs[idx], None, None, None
        _mm_add_body(refs[0], refs[1], a_ref, b_ref, y_ref, st_ref,
                     ssum_ref, ssq_ref, nblocks=nblocks,
                     has_addend=has_addend, with_stats=with_stats)

    if with_stats:
        st_spec = pl.BlockSpec((2, Fout), lambda i: (0, 0))
        y, st = pl.pallas_call(
            body,
            grid=(nblocks,),
            in_specs=in_specs,
            out_specs=[y_spec, st_spec],
            out_shape=[jax.ShapeDtypeStruct((V, Fout), jnp.float32),
                       jax.ShapeDtypeStruct((2, Fout), jnp.float32)],
            scratch_shapes=[pltpu.VMEM((1, Fout), jnp.float32),
                            pltpu.VMEM((1, Fout), jnp.float32)],
        )(*args)
        return y, st
    y = pl.pallas_call(
        body,
        grid=(nblocks,),
        in_specs=in_specs,
        out_specs=y_spec,
        out_shape=jax.ShapeDtypeStruct((V, Fout), jnp.float32),
    )(*args)
    return y, None


def _cheb_combine(x0, g1, g2, w_stack, b, with_stats):
    V, Fin = x0.shape
    Fout = w_stack.shape[1]
    BV = _row_block(V)
    nblocks = V // BV
    b2 = b.reshape(1, Fout)
    row_spec = pl.BlockSpec((BV, Fin), lambda i: (i, 0))
    w_spec = pl.BlockSpec((3 * Fin, Fout), lambda i: (0, 0))
    b_spec = pl.BlockSpec((1, Fout), lambda i: (0, 0))
    y_spec = pl.BlockSpec((BV, Fout), lambda i: (i, 0))
    if with_stats:
        st_spec = pl.BlockSpec((2, Fout), lambda i: (0, 0))
        y, st = pl.pallas_call(
            functools.partial(_mm_body, Fin=Fin, nblocks=nblocks),
            grid=(nblocks,),
            in_specs=[row_spec, row_spec, row_spec, w_spec, b_spec],
            out_specs=[y_spec, st_spec],
            out_shape=[jax.ShapeDtypeStruct((V, Fout), jnp.float32),
                       jax.ShapeDtypeStruct((2, Fout), jnp.float32)],
            scratch_shapes=[pltpu.VMEM((1, Fout), jnp.float32),
                            pltpu.VMEM((1, Fout), jnp.float32)],
        )(x0, g1, g2, w_stack, b2)
        return y, st
    y = pl.pallas_call(
        functools.partial(_mm_plain_body, Fin=Fin),
        grid=(nblocks,),
        in_specs=[row_spec, row_spec, row_spec, w_spec, b_spec],
        out_specs=y_spec,
        out_shape=jax.ShapeDtypeStruct((V, Fout), jnp.float32),
    )(x0, g1, g2, w_stack, b2)
    return y, None


def _combine_bn_body(x0_ref, g1_ref, g2_ref, w_ref, b_ref, gb_ref,
                     z_ref, p_ref, y_sc, ssum_ref, ssq_ref,
                     *, Fin, nblocks, V, BV, pool):
    i = pl.program_id(0)

    @pl.when(i == 0)
    def _init():
        ssum_ref[...] = jnp.zeros_like(ssum_ref)
        ssq_ref[...] = jnp.zeros_like(ssq_ref)

    @pl.when(i < nblocks)
    def _mm():
        y = (_dot(x0_ref[...], w_ref[:Fin, :])
             + _dot(g1_ref[...], w_ref[Fin:2 * Fin, :])
             + _dot(g2_ref[...], w_ref[2 * Fin:, :])
             + b_ref[...])
        y_sc[pl.ds(i * BV, BV), :] = y
        ssum_ref[...] += jnp.sum(y, axis=0, keepdims=True)
        ssq_ref[...] += jnp.sum(y * y, axis=0, keepdims=True)

    @pl.when(i >= nblocks)
    def _bn():
        y = y_sc[pl.ds((i - nblocks) * BV, BV), :]
        m = ssum_ref[...] / V
        var = ssq_ref[...] / V - m * m
        inv = lax.rsqrt(var + _EPS)
        z = jnp.maximum((y - m) * (inv * gb_ref[0:1, :]) + gb_ref[1:2, :],
                        0.0)
        z_ref[...] = z
        if pool:
            f = z.shape[1]
            p_ref[...] = jnp.mean(z.reshape(BV // 4, 4, f), axis=1)


def _combine_bn(x0, g1, g2, w_stack, b, g, be, pool):
    V, Fin = x0.shape
    Fout = w_stack.shape[1]
    BV = _row_block(V)
    n = V // BV
    gb = jnp.stack([g, be], axis=0)

    def row_map(i, nb=n):
        return (jnp.where(i < nb, i, 0), 0)

    def out_map(i, nb=n):
        return (jnp.where(i < nb, 0, i - nb), 0)

    in_specs = [pl.BlockSpec((BV, Fin), row_map),
                pl.BlockSpec((BV, Fin), row_map),
                pl.BlockSpec((BV, Fin), row_map),
                pl.BlockSpec((3 * Fin, Fout), lambda i: (0, 0)),
                pl.BlockSpec((1, Fout), lambda i: (0, 0)),
                pl.BlockSpec((2, Fout), lambda i: (0, 0))]
    out_specs = [pl.BlockSpec((BV, Fout), out_map)]
    out_shape = [jax.ShapeDtypeStruct((V, Fout), jnp.float32)]
    if pool:
        out_specs.append(pl.BlockSpec((BV // 4, Fout), out_map))
        out_shape.append(jax.ShapeDtypeStruct((V // 4, Fout), jnp.float32))
    def body(*refs):
        if pool:
            (x0_r, g1_r, g2_r, w_r, b_r, gb_r, z_r, p_r, y_sc, s1, s2) = refs
        else:
            (x0_r, g1_r, g2_r, w_r, b_r, gb_r, z_r, y_sc, s1, s2) = refs
            p_r = None
        _combine_bn_body(x0_r, g1_r, g2_r, w_r, b_r, gb_r, z_r, p_r,
                         y_sc, s1, s2, Fin=Fin, nblocks=n, V=V, BV=BV,
                         pool=pool)

    res = pl.pallas_call(
        body,
        grid=(2 * n,),
        in_specs=in_specs,
        out_specs=out_specs if pool else out_specs[0],
        out_shape=out_shape if pool else out_shape[0],
        scratch_shapes=[pltpu.VMEM((V, Fout), jnp.float32),
                        pltpu.VMEM((1, Fout), jnp.float32),
                        pltpu.VMEM((1, Fout), jnp.float32)],
    )(x0, g1, g2, w_stack, b.reshape(1, Fout), gb)
    return res


def _bn_body(y_ref, st_ref, gb_ref, z_ref, *, V):
    m = st_ref[0:1, :] / V
    var = st_ref[1:2, :] / V - m * m
    inv = lax.rsqrt(var + _EPS)
    z_ref[...] = jnp.maximum(
        (y_ref[...] - m) * (inv * gb_ref[0:1, :]) + gb_ref[1:2, :], 0.0)


def _bn_pool_body(y_ref, st_ref, gb_ref, z_ref, p_ref, *, V):
    m = st_ref[0:1, :] / V
    var = st_ref[1:2, :] / V - m * m
    inv = lax.rsqrt(var + _EPS)
    z = jnp.maximum(
        (y_ref[...] - m) * (inv * gb_ref[0:1, :]) + gb_ref[1:2, :], 0.0)
    z_ref[...] = z
    bv, f = z.shape
    p_ref[...] = jnp.mean(z.reshape(bv // 4, 4, f), axis=1)


def _bn_relu(y, st, g, be, pool=False):
    V, F = y.shape
    BV = _row_block(V)
    gb = jnp.stack([g, be], axis=0)
    in_specs = [pl.BlockSpec((BV, F), lambda i: (i, 0)),
                pl.BlockSpec((2, F), lambda i: (0, 0)),
                pl.BlockSpec((2, F), lambda i: (0, 0))]
    if not pool:
        return pl.pallas_call(
            functools.partial(_bn_body, V=V),
            grid=(V // BV,),
            in_specs=in_specs,
            out_specs=pl.BlockSpec((BV, F), lambda i: (i, 0)),
            out_shape=jax.ShapeDtypeStruct((V, F), jnp.float32),
        )(y, st, gb)
    return pl.pallas_call(
        functools.partial(_bn_pool_body, V=V),
        grid=(V // BV,),
        in_specs=in_specs,
        out_specs=[pl.BlockSpec((BV, F), lambda i: (i, 0)),
                   pl.BlockSpec((BV // 4, F), lambda i: (i, 0))],
        out_shape=[jax.ShapeDtypeStruct((V, F), jnp.float32),
                   jax.ShapeDtypeStruct((V // 4, F), jnp.float32)],
    )(y, st, gb)


def _pool_body(x_ref, p_ref):
    p_ref[...] = jnp.mean(x_ref[...], axis=1)


def _pool(x):
    V, F = x.shape
    Vp = V // 4
    BP = _row_block(Vp)
    x3 = x.reshape(Vp, 4, F)
    return pl.pallas_call(
        _pool_body,
        grid=(Vp // BP,),
        in_specs=[pl.BlockSpec((BP, 4, F), lambda i: (i, 0, 0))],
        out_specs=pl.BlockSpec((BP, F), lambda i: (i, 0)),
        out_shape=jax.ShapeDtypeStruct((Vp, F), jnp.float32),
    )(x3)


def _unpool_body(x_ref, u_ref):
    b, _, f = u_ref.shape
    u_ref[...] = jnp.broadcast_to(x_ref[...][:, None, :], (b, 4, f))


def _unpool(x):
    Vc, F = x.shape
    BP = _row_block(Vc)
    u = pl.pallas_call(
        _unpool_body,
        grid=(Vc // BP,),
        in_specs=[pl.BlockSpec((BP, F), lambda i: (i, 0))],
        out_specs=pl.BlockSpec((BP, 4, F), lambda i: (i, 0, 0)),
        out_shape=jax.ShapeDtypeStruct((Vc, 4, F), jnp.float32),
    )(x)
    return u.reshape(4 * Vc, F)


def _prep_weights(w):
    K, Fin, Fout = w.shape
    weff = jnp.transpose(w.reshape(Fin, K, Fout), (1, 0, 2))
    return weff[0] - weff[2], weff[1], 2.0 * weff[2]


def _spmm_chain(lap, x0):
    if x0.shape[0] <= _SMALL_V:
        return _spmm_pair(lap, x0)
    g1 = _spmm(lap, x0)
    return g1, _spmm(lap, g1)


def _cheb_fin(lap, x0, p, name, with_stats):
    wa, wb, wc = _prep_weights(p[name + '_w'])
    w_stack = jnp.concatenate([wa, wb, wc], axis=0)
    g1, g2 = _spmm_chain(lap, x0)
    return _cheb_combine(x0, g1, g2, w_stack, p[name + '_b'], with_stats)


def _fout_weights(p, name):
    wa, wb, wc = _prep_weights(p[name + '_w'])
    b = p[name + '_b']
    Fout = wa.shape[1]
    wcat = jnp.concatenate([wc, wb, wa], axis=1)
    bias3 = jnp.concatenate(
        [jnp.zeros((2 * Fout,), jnp.float32), b]).reshape(1, 3 * Fout)
    return wcat, bias3


def _cheb_fout_x(lap, x0, p, name):
    wcat, bias3 = _fout_weights(p, name)
    q, a2, a3 = _mm3_x(x0, wcat, bias3)
    if q.shape[0] <= _SMALL_V:
        return _spmm_pair_add(lap, q, a2, a3)[1]
    t = _spmm(lap, q, addend=a2)
    return _spmm(lap, t, addend=a3)


def _cheb_fout_up(lap, zc, skip, p, name):
    wcat, bias3 = _fout_weights(p, name)
    Fu = zc.shape[1]
    q, a2, a3 = _mm3_up(zc, skip, wcat[:Fu], wcat[Fu:], bias3)
    if q.shape[0] <= _SMALL_V:
        return _spmm_pair_add(lap, q, a2, a3)[1]
    t = _spmm(lap, q, addend=a2)
    return _spmm(lap, t, addend=a3)


def _cbn_fin(lap, x0, p, name, pool=False):
    wa, wb, wc = _prep_weights(p[name + '_w'])
    w_stack = jnp.concatenate([wa, wb, wc], axis=0)
    g1, g2 = _spmm_chain(lap, x0)
    return _combine_bn(x0, g1, g2, w_stack, p[name + '_b'],
                       p[name + '_g'], p[name + '_be'], pool)


def kernel(x, params, laps):
    p = params
    x0 = x[0]

    x5a = _cbn_fin(laps[5], x0, p, 'e5a')
    x5, x5p = _cbn_fin(laps[5], x5a, p, 'e5b', pool=True)
    x4, x4p = _cbn_fin(laps[4], x5p, p, 'e4', pool=True)
    x3, x3p = _cbn_fin(laps[3], x4p, p, 'e3', pool=True)
    x2, x2p = _cbn_fin(laps[2], x3p, p, 'e2', pool=True)
    _, x1p = _cbn_fin(laps[1], x2p, p, 'e1', pool=True)
    xb = _cheb_fin(laps[0], x1p, p, 'e0', False)[0]

    z1 = _cbn_fin(laps[1], _unpool(xb), p, 'd1a')
    d1 = _cheb_fout_x(laps[1], z1, p, 'd1b')
    d2 = _stats_bn(_cheb_fout_up(laps[2], d1, x2, p, 'd2'),
                   p['d2_g'], p['d2_be'])
    d3 = _stats_bn(_cheb_fout_up(laps[3], d2, x3, p, 'd3'),
                   p['d3_g'], p['d3_be'])
    d4 = _stats_bn(_cheb_fout_up(laps[4], d3, x4, p, 'd4'),
                   p['d4_g'], p['d4_be'])
    d5 = _stats_bn(_cheb_fout_up(laps[5], d4, x5, p, 'd5'),
                   p['d5_g'], p['d5_be'])
    out = _cheb_fin(laps[5], d5, p, 'df', False)[0]
    return out.reshape(1, out.shape[0], 1)


# speedup vs baseline: 1.0249x; 1.0085x over previous
import functools

import jax
import jax.numpy as jnp
from jax import lax
from jax.experimental import pallas as pl
from jax.experimental.pallas import tpu as pltpu
from jax.experimental.pallas import tpu_sc as plsc

K_CHEB = 3
_EPS = 1e-5

_NC, _NS = 2, 16
_NW = _NC * _NS


def _pick_chunk(rpw, F):
    def best_le(limit):
        best = 0
        c = 8
        while c <= limit:
            if rpw % c == 0 and 32 * c * F <= 150_000:
                best = c
            c += 8
        return best

    c = best_le(rpw // 2)
    if c == 0:
        c = best_le(rpw)
    return c if c else 8


def _splat(wv, k):
    return lax.gather(
        wv, jnp.full((16, 1), k, jnp.int32),
        lax.GatherDimensionNumbers(
            offset_dims=(), collapsed_slice_dims=(0,),
            start_index_map=(0,)),
        (1,),
        mode=lax.GatherScatterMode.PROMISE_IN_BOUNDS)


@functools.lru_cache(maxsize=None)
def _make_spmm(V, F, with_add):
    nw = min(_NW, V // 8)
    rpw = V // nw
    C = _pick_chunk(rpw, F)
    nj = rpw // C
    NF = F // 16
    nbuf = 2 if nj >= 2 else 1

    mesh = plsc.VectorSubcoreMesh(core_axis_name="c", subcore_axis_name="s")

    buf_types = []
    for _ in range(nbuf):
        buf_types += [
            pltpu.VMEM((C, 16), jnp.float32),
            pltpu.VMEM((8 * C,), jnp.int32),
            pltpu.VMEM((8 * C, F), jnp.float32),
            pltpu.VMEM((C, F), jnp.float32),
            pltpu.VMEM((C, F), jnp.float32),
            pltpu.VMEM((C, F), jnp.float32),
            pltpu.SemaphoreType.DMA,
            pltpu.SemaphoreType.DMA,
            pltpu.SemaphoreType.DMA,
        ]

    def spmm_kernel(*args):
            if with_add:
                x_hbm, cols_hbm, vals_hbm, add_hbm, y_hbm = args[:5]
                scratch = args[5:]
            else:
                x_hbm, cols_hbm, vals_hbm, y_hbm = args[:4]
                add_hbm = None
                scratch = args[4:]
            bufs = [scratch[9 * i:9 * i + 9] for i in range(nbuf)]
            wid = lax.axis_index("s") * _NC + lax.axis_index("c")

            def stage1(buf, j):
                vals_v, idx_v, _, xs_v, _, add_v, sem_in, _, _ = buf
                base = wid * rpw + j * C
                pltpu.async_copy(vals_hbm.at[pl.ds(base, C), :], vals_v,
                                 sem_in)
                pltpu.async_copy(cols_hbm.at[pl.ds(8 * base, 8 * C)], idx_v,
                                 sem_in)
                pltpu.async_copy(x_hbm.at[pl.ds(base, C), :], xs_v, sem_in)
                if with_add:
                    pltpu.async_copy(add_hbm.at[pl.ds(base, C), :], add_v,
                                     sem_in)

            def wait_stage1(buf, j):
                vals_v, idx_v, _, xs_v, _, add_v, sem_in, _, _ = buf
                base = wid * rpw + j * C
                pltpu.make_async_copy(vals_hbm.at[pl.ds(base, C), :], vals_v,
                                      sem_in).wait()
                pltpu.make_async_copy(cols_hbm.at[pl.ds(8 * base, 8 * C)],
                                      idx_v, sem_in).wait()
                pltpu.make_async_copy(x_hbm.at[pl.ds(base, C), :], xs_v,
                                      sem_in).wait()
                if with_add:
                    pltpu.make_async_copy(add_hbm.at[pl.ds(base, C), :],
                                          add_v, sem_in).wait()

            def gather(buf):
                _, idx_v, rows_v, _, _, _, _, sem_g, _ = buf
                pltpu.async_copy(x_hbm.at[idx_v], rows_v, sem_g)

            def wait_gather(buf):
                _, idx_v, rows_v, _, _, _, _, sem_g, _ = buf
                pltpu.make_async_copy(x_hbm.at[idx_v], rows_v, sem_g).wait()

            def put_y(buf, j):
                y_v, sem_y = buf[4], buf[8]
                base = wid * rpw + j * C
                pltpu.async_copy(y_v, y_hbm.at[pl.ds(base, C), :], sem_y)

            def wait_y(buf, j):
                y_v, sem_y = buf[4], buf[8]
                base = wid * rpw + j * C
                pltpu.make_async_copy(y_v, y_hbm.at[pl.ds(base, C), :],
                                      sem_y).wait()

            def compute(buf):
                vals_v, _, rows_v, xs_v, y_v, add_v, _, _, _ = buf

                def row(r, carry):
                    wv = vals_v[r, :]
                    ws = [_splat(wv, k) for k in range(8)]
                    dw = _splat(wv, 8)
                    for f in range(NF):
                        sl = pl.ds(16 * f, 16)
                        acc = dw * xs_v[r, sl]
                        if with_add:
                            acc = acc + add_v[r, sl]
                        for k in range(8):
                            acc = acc + ws[k] * rows_v[8 * r + k, sl]
                        y_v[r, sl] = acc
                    return carry

                lax.fori_loop(0, C, row, 0)

            @pl.when(wid < nw)
            def _work():
                if nbuf == 1:
                    buf = bufs[0]

                    def chunk(j, carry):
                        stage1(buf, j)
                        wait_stage1(buf, j)
                        gather(buf)
                        wait_gather(buf)
                        compute(buf)
                        base = wid * rpw + j * C
                        pltpu.sync_copy(buf[4], y_hbm.at[pl.ds(base, C), :])
                        return carry

                    lax.fori_loop(0, nj, chunk, 0)
                else:
                    b0, b1 = bufs
                    stage1(b0, 0)
                    stage1(b1, 1)
                    wait_stage1(b0, 0)
                    gather(b0)

                    def pair(t, carry):
                        j0 = 2 * t
                        @pl.when(t > 0)
                        def _():
                            wait_y(b0, j0 - 2)

                        @pl.when(j0 + 1 < nj)
                        def _():
                            wait_stage1(b1, j0 + 1)
                            gather(b1)
                        wait_gather(b0)
                        compute(b0)
                        put_y(b0, j0)

                        @pl.when(j0 + 2 < nj)
                        def _():
                            stage1(b0, j0 + 2)

                        @pl.when(j0 + 1 < nj)
                        def _():
                            @pl.when(t > 0)
                            def _():
                                wait_y(b1, j0 - 1)

                            @pl.when(j0 + 2 < nj)
                            def _():
                                wait_stage1(b0, j0 + 2)
                                gather(b0)
                            wait_gather(b1)
                            compute(b1)
                            put_y(b1, j0 + 1)

                            @pl.when(j0 + 3 < nj)
                            def _():
                                stage1(b1, j0 + 3)
                        return carry

                    lax.fori_loop(0, (nj + 1) // 2, pair, 0)
                    wait_y(b0, 0)
                    wait_y(b1, 0)

    return pl.kernel(
        spmm_kernel,
        out_type=jax.ShapeDtypeStruct((V, F), jnp.float32),
        mesh=mesh,
        scratch_types=buf_types,
        compiler_params=pltpu.CompilerParams(use_tc_tiling_on_sc=False),
    )


@functools.lru_cache(maxsize=None)
def _make_spmm2(V, F, with_add):
    nw = min(16, V // 8)
    rpw = V // nw
    C = rpw
    c = 1
    while c <= rpw:
        if rpw % c == 0 and 32 * c * F <= 100_000:
            C = c
        c += 1
    nj = rpw // C
    NF = F // 16

    mesh = plsc.VectorSubcoreMesh(core_axis_name="c", subcore_axis_name="s",
                                  num_cores=1)

    scratch = [
        pltpu.VMEM((C, 16), jnp.float32),
        pltpu.VMEM((8 * C,), jnp.int32),
        pltpu.VMEM((8 * C, F), jnp.float32),
        pltpu.VMEM((C, F), jnp.float32),
        pltpu.VMEM((C, F), jnp.float32),
        pltpu.VMEM((C, F), jnp.float32),
        pltpu.SemaphoreType.DMA,
    ]

    def spmm2_kernel(*args):
        if with_add:
            x_hbm, cols_hbm, vals_hbm, a2_hbm, a3_hbm, o1_hbm, o2_hbm = \
                args[:7]
            rest = args[7:]
        else:
            x_hbm, cols_hbm, vals_hbm, o1_hbm, o2_hbm = args[:5]
            a2_hbm = a3_hbm = None
            rest = args[5:]
        vals_v, idx_v, rows_v, xs_v, y_v, add_v, sem = rest
        wid = lax.axis_index("s")

        def phase(src_hbm, dst_hbm, add_hbm):
            def chunk(j, carry):
                base = wid * rpw + j * C
                pltpu.sync_copy(vals_hbm.at[pl.ds(base, C), :], vals_v)
                pltpu.sync_copy(cols_hbm.at[pl.ds(8 * base, 8 * C)], idx_v)
                pltpu.sync_copy(src_hbm.at[pl.ds(base, C), :], xs_v)
                if add_hbm is not None:
                    pltpu.sync_copy(add_hbm.at[pl.ds(base, C), :], add_v)
                pltpu.async_copy(src_hbm.at[idx_v], rows_v, sem).wait()

                def row(r, carry2):
                    wv = vals_v[r, :]
                    ws = [_splat(wv, k) for k in range(8)]
                    dw = _splat(wv, 8)
                    for f in range(NF):
                        sl = pl.ds(16 * f, 16)
                        acc = dw * xs_v[r, sl]
                        if add_hbm is not None:
                            acc = acc + add_v[r, sl]
                        for k in range(8):
                            acc = acc + ws[k] * rows_v[8 * r + k, sl]
                        y_v[r, sl] = acc
                    return carry2

                lax.fori_loop(0, C, row, 0)
                pltpu.sync_copy(y_v, dst_hbm.at[pl.ds(base, C), :])
                return carry

            lax.fori_loop(0, nj, chunk, 0)

        @pl.when(wid < nw)
        def _p1():
            phase(x_hbm, o1_hbm, a2_hbm)

        plsc.subcore_barrier()

        @pl.when(wid < nw)
        def _p2():
            phase(o1_hbm, o2_hbm, a3_hbm)

    out = jax.ShapeDtypeStruct((V, F), jnp.float32)
    return pl.kernel(
        spmm2_kernel,
        out_type=(out, out),
        mesh=mesh,
        scratch_types=scratch,
        compiler_params=pltpu.CompilerParams(use_tc_tiling_on_sc=False),
    )


_SMALL_V = 0


def _spmm_pair(lap, x):
    V, F = x.shape
    cols8, vals16 = _lap_prep(lap, V)
    return _make_spmm2(V, F, False)(x, cols8, vals16)


def _spmm_pair_add(lap, q, a2, a3):
    V, F = q.shape
    cols8, vals16 = _lap_prep(lap, V)
    return _make_spmm2(V, F, True)(q, cols8, vals16, a2, a3)


def _lap_prep(lap, V):
    _, cols, vals = lap
    e = 8 * V
    vals16 = jnp.concatenate(
        [vals[:e].reshape(V, 8), vals[e:, None],
         jnp.zeros((V, 7), jnp.float32)], axis=1)
    return cols[:e], vals16


def _spmm(lap, x, addend=None):
    V, F = x.shape
    cols8, vals16 = _lap_prep(lap, V)
    if addend is None:
        return _make_spmm(V, F, False)(x, cols8, vals16)
    return _make_spmm(V, F, True)(x, cols8, vals16, addend)


def _row_block(V):
    return 256 if V % 256 == 0 else V


def _dot(a, b):
    return jnp.dot(a, b, preferred_element_type=jnp.float32,
                   precision=lax.Precision.HIGHEST)


def _mm_body(x0_ref, g1_ref, g2_ref, w_ref, b_ref, y_ref, st_ref,
             ssum_ref, ssq_ref, *, Fin, nblocks):
    i = pl.program_id(0)
    y = (_dot(x0_ref[...], w_ref[:Fin, :])
         + _dot(g1_ref[...], w_ref[Fin:2 * Fin, :])
         + _dot(g2_ref[...], w_ref[2 * Fin:, :])
         + b_ref[...])
    y_ref[...] = y

    @pl.when(i == 0)
    def _init():
        ssum_ref[...] = jnp.zeros_like(ssum_ref)
        ssq_ref[...] = jnp.zeros_like(ssq_ref)

    ssum_ref[...] += jnp.sum(y, axis=0, keepdims=True)
    ssq_ref[...] += jnp.sum(y * y, axis=0, keepdims=True)

    @pl.when(i == nblocks - 1)
    def _fin():
        st_ref[...] = jnp.concatenate([ssum_ref[...], ssq_ref[...]], axis=0)


def _mm_plain_body(x0_ref, g1_ref, g2_ref, w_ref, b_ref, y_ref, *, Fin):
    y_ref[...] = (
        _dot(x0_ref[...], w_ref[:Fin, :])
        + _dot(g1_ref[...], w_ref[Fin:2 * Fin, :])
        + _dot(g2_ref[...], w_ref[2 * Fin:, :])
        + b_ref[...])


def _mm3_x_body(x_ref, w_ref, b_ref, q_ref, a2_ref, a3_ref, *, Fout):
    P = _dot(x_ref[...], w_ref[...]) + b_ref[...]
    q_ref[...] = P[:, :Fout]
    a2_ref[...] = P[:, Fout:2 * Fout]
    a3_ref[...] = P[:, 2 * Fout:]


def _mm3_x(x, wcat, bias3):
    V, Fin = x.shape
    Fout = wcat.shape[1] // 3
    BV = _row_block(V)
    nblocks = V // BV
    out_spec = pl.BlockSpec((BV, Fout), lambda i: (i, 0))
    out_shape = jax.ShapeDtypeStruct((V, Fout), jnp.float32)
    return pl.pallas_call(
        functools.partial(_mm3_x_body, Fout=Fout),
        grid=(nblocks,),
        in_specs=[pl.BlockSpec((BV, Fin), lambda i: (i, 0)),
                  pl.BlockSpec((Fin, 3 * Fout), lambda i: (0, 0)),
                  pl.BlockSpec((1, 3 * Fout), lambda i: (0, 0))],
        out_specs=[out_spec, out_spec, out_spec],
        out_shape=[out_shape, out_shape, out_shape],
    )(x, wcat, bias3)


def _mm3_up_body(zc_ref, s_ref, wu_ref, ws_ref, b_ref,
                 q_ref, a2_ref, a3_ref, *, Fout):
    bv4 = zc_ref.shape[0]
    pu = _dot(zc_ref[...], wu_ref[...])
    pu4 = jnp.broadcast_to(pu[:, None, :], (bv4, 4, 3 * Fout))
    pu4 = pu4.reshape(4 * bv4, 3 * Fout)
    P = pu4 + _dot(s_ref[...], ws_ref[...]) + b_ref[...]
    q_ref[...] = P[:, :Fout]
    a2_ref[...] = P[:, Fout:2 * Fout]
    a3_ref[...] = P[:, 2 * Fout:]


def _mm3_up(zc, skip, wu, ws, bias3):
    Vc, Fu = zc.shape
    V, Fs = skip.shape
    Fout = wu.shape[1] // 3
    BV = _row_block(V)
    nblocks = V // BV
    out_spec = pl.BlockSpec((BV, Fout), lambda i: (i, 0))
    out_shape = jax.ShapeDtypeStruct((V, Fout), jnp.float32)
    return pl.pallas_call(
        functools.partial(_mm3_up_body, Fout=Fout),
        grid=(nblocks,),
        in_specs=[pl.BlockSpec((BV // 4, Fu), lambda i: (i, 0)),
                  pl.BlockSpec((BV, Fs), lambda i: (i, 0)),
                  pl.BlockSpec((Fu, 3 * Fout), lambda i: (0, 0)),
                  pl.BlockSpec((Fs, 3 * Fout), lambda i: (0, 0)),
                  pl.BlockSpec((1, 3 * Fout), lambda i: (0, 0))],
        out_specs=[out_spec, out_spec, out_spec],
        out_shape=[out_shape, out_shape, out_shape],
    )(zc, skip, wu, ws, bias3)


def _stats_bn_body(y_ref, gb_ref, z_ref, ssum_ref, ssq_ref, *, V, nblocks):
    i = pl.program_id(0)

    @pl.when(i == 0)
    def _init():
        ssum_ref[...] = jnp.zeros_like(ssum_ref)
        ssq_ref[...] = jnp.zeros_like(ssq_ref)

    y = y_ref[...]

    @pl.when(i < nblocks)
    def _acc():
        ssum_ref[...] += jnp.sum(y, axis=0, keepdims=True)
        ssq_ref[...] += jnp.sum(y * y, axis=0, keepdims=True)
        z_ref[...] = y

    @pl.when(i >= nblocks)
    def _apply():
        m = ssum_ref[...] / V
        var = ssq_ref[...] / V - m * m
        inv = lax.rsqrt(var + _EPS)
        z_ref[...] = jnp.maximum((y - m) * (inv * gb_ref[0:1, :])
                                 + gb_ref[1:2, :], 0.0)


def _stats_bn(y, g, be):
    V, F = y.shape
    BV = _row_block(V)
    nblocks = V // BV
    gb = jnp.stack([g, be], axis=0)
    return pl.pallas_call(
        functools.partial(_stats_bn_body, V=V, nblocks=nblocks),
        grid=(2 * nblocks,),
        in_specs=[pl.BlockSpec((BV, F), lambda i, n=nblocks: (i % n, 0)),
                  pl.BlockSpec((2, F), lambda i: (0, 0))],
        out_specs=pl.BlockSpec((BV, F), lambda i, n=nblocks: (i % n, 0)),
        out_shape=jax.ShapeDtypeStruct((V, F), jnp.float32),
        scratch_shapes=[pltpu.VMEM((1, F), jnp.float32),
                        pltpu.VMEM((1, F), jnp.float32)],
    )(y, gb)


def _mm_add_body(x_ref, w_ref, a_ref, b_ref, y_ref, st_ref,
                 ssum_ref, ssq_ref, *, nblocks, has_addend, with_stats):
    i = pl.program_id(0)
    y = _dot(x_ref[...], w_ref[...])
    if has_addend:
        y = y + a_ref[...]
    if b_ref is not None:
        y = y + b_ref[...]
    y_ref[...] = y
    if with_stats:
        @pl.when(i == 0)
        def _init():
            ssum_ref[...] = jnp.zeros_like(ssum_ref)
            ssq_ref[...] = jnp.zeros_like(ssq_ref)

        ssum_ref[...] += jnp.sum(y, axis=0, keepdims=True)
        ssq_ref[...] += jnp.sum(y * y, axis=0, keepdims=True)

        @pl.when(i == nblocks - 1)
        def _fin():
            st_ref[...] = jnp.concatenate([ssum_ref[...], ssq_ref[...]],
                                          axis=0)


def _mm_add(x, w, addend=None, bias=None, with_stats=False):
    V, Fin = x.shape
    Fout = w.shape[1]
    BV = _row_block(V)
    nblocks = V // BV
    row_spec = pl.BlockSpec((BV, Fin), lambda i: (i, 0))
    w_spec = pl.BlockSpec((Fin, Fout), lambda i: (0, 0))
    vec_spec = pl.BlockSpec((1, Fout), lambda i: (0, 0))
    y_spec = pl.BlockSpec((BV, Fout), lambda i: (i, 0))
    a_spec = pl.BlockSpec((BV, Fout), lambda i: (i, 0))
    args = [x, w]
    in_specs = [row_spec, w_spec]
    has_addend = addend is not None
    if has_addend:
        args.append(addend)
        in_specs.append(a_spec)

    if bias is not None:
        args.append(bias.reshape(1, Fout))
        in_specs.append(vec_spec)

    def body(*refs):
        idx = 2
        a_ref = None
        b_ref = None
        if has_addend:
            a_ref = refs[idx]; idx += 1
        if bias is not None:
            b_ref = refs[idx]; idx += 1
        if with_stats:
            y_ref, st_ref = refs[idx], refs[idx + 1]
            ssum_ref, ssq_ref = refs[idx + 2], refs[idx + 3]
        else:
            y_ref, st_ref, ssum_ref, ssq_ref = refs[idx], None, None, None
        _mm_add_body(refs[0], refs[1], a_ref, b_ref, y_ref, st_ref,
                     ssum_ref, ssq_ref, nblocks=nblocks,
                     has_addend=has_addend, with_stats=with_stats)

    if with_stats:
        st_spec = pl.BlockSpec((2, Fout), lambda i: (0, 0))
        y, st = pl.pallas_call(
            body,
            grid=(nblocks,),
            in_specs=in_specs,
            out_specs=[y_spec, st_spec],
            out_shape=[jax.ShapeDtypeStruct((V, Fout), jnp.float32),
                       jax.ShapeDtypeStruct((2, Fout), jnp.float32)],
            scratch_shapes=[pltpu.VMEM((1, Fout), jnp.float32),
                            pltpu.VMEM((1, Fout), jnp.float32)],
        )(*args)
        return y, st
    y = pl.pallas_call(
        body,
        grid=(nblocks,),
        in_specs=in_specs,
        out_specs=y_spec,
        out_shape=jax.ShapeDtypeStruct((V, Fout), jnp.float32),
    )(*args)
    return y, None


def _cheb_combine(x0, g1, g2, w_stack, b, with_stats):
    V, Fin = x0.shape
    Fout = w_stack.shape[1]
    BV = _row_block(V)
    nblocks = V // BV
    b2 = b.reshape(1, Fout)
    row_spec = pl.BlockSpec((BV, Fin), lambda i: (i, 0))
    w_spec = pl.BlockSpec((3 * Fin, Fout), lambda i: (0, 0))
    b_spec = pl.BlockSpec((1, Fout), lambda i: (0, 0))
    y_spec = pl.BlockSpec((BV, Fout), lambda i: (i, 0))
    if with_stats:
        st_spec = pl.BlockSpec((2, Fout), lambda i: (0, 0))
        y, st = pl.pallas_call(
            functools.partial(_mm_body, Fin=Fin, nblocks=nblocks),
            grid=(nblocks,),
            in_specs=[row_spec, row_spec, row_spec, w_spec, b_spec],
            out_specs=[y_spec, st_spec],
            out_shape=[jax.ShapeDtypeStruct((V, Fout), jnp.float32),
                       jax.ShapeDtypeStruct((2, Fout), jnp.float32)],
            scratch_shapes=[pltpu.VMEM((1, Fout), jnp.float32),
                            pltpu.VMEM((1, Fout), jnp.float32)],
        )(x0, g1, g2, w_stack, b2)
        return y, st
    y = pl.pallas_call(
        functools.partial(_mm_plain_body, Fin=Fin),
        grid=(nblocks,),
        in_specs=[row_spec, row_spec, row_spec, w_spec, b_spec],
        out_specs=y_spec,
        out_shape=jax.ShapeDtypeStruct((V, Fout), jnp.float32),
    )(x0, g1, g2, w_stack, b2)
    return y, None


def _combine_bn_body(x0_ref, g1_ref, g2_ref, w_ref, b_ref, gb_ref,
                     z_ref, p_ref, y_sc, ssum_ref, ssq_ref,
                     *, Fin, nblocks, V, BV, pool):
    i = pl.program_id(0)

    @pl.when(i == 0)
    def _init():
        ssum_ref[...] = jnp.zeros_like(ssum_ref)
        ssq_ref[...] = jnp.zeros_like(ssq_ref)

    @pl.when(i < nblocks)
    def _mm():
        y = (_dot(x0_ref[...], w_ref[:Fin, :])
             + _dot(g1_ref[...], w_ref[Fin:2 * Fin, :])
             + _dot(g2_ref[...], w_ref[2 * Fin:, :])
             + b_ref[...])
        y_sc[pl.ds(i * BV, BV), :] = y
        ssum_ref[...] += jnp.sum(y, axis=0, keepdims=True)
        ssq_ref[...] += jnp.sum(y * y, axis=0, keepdims=True)

    @pl.when(i >= nblocks)
    def _bn():
        y = y_sc[pl.ds((i - nblocks) * BV, BV), :]
        m = ssum_ref[...] / V
        var = ssq_ref[...] / V - m * m
        inv = lax.rsqrt(var + _EPS)
        z = jnp.maximum((y - m) * (inv * gb_ref[0:1, :]) + gb_ref[1:2, :],
                        0.0)
        z_ref[...] = z
        if pool:
            f = z.shape[1]
            p_ref[...] = jnp.mean(z.reshape(BV // 4, 4, f), axis=1)


def _combine_bn(x0, g1, g2, w_stack, b, g, be, pool):
    V, Fin = x0.shape
    Fout = w_stack.shape[1]
    BV = _row_block(V)
    n = V // BV
    gb = jnp.stack([g, be], axis=0)

    def row_map(i, nb=n):
        return (jnp.where(i < nb, i, 0), 0)

    def out_map(i, nb=n):
        return (jnp.where(i < nb, 0, i - nb), 0)

    in_specs = [pl.BlockSpec((BV, Fin), row_map),
                pl.BlockSpec((BV, Fin), row_map),
                pl.BlockSpec((BV, Fin), row_map),
                pl.BlockSpec((3 * Fin, Fout), lambda i: (0, 0)),
                pl.BlockSpec((1, Fout), lambda i: (0, 0)),
                pl.BlockSpec((2, Fout), lambda i: (0, 0))]
    out_specs = [pl.BlockSpec((BV, Fout), out_map)]
    out_shape = [jax.ShapeDtypeStruct((V, Fout), jnp.float32)]
    if pool:
        out_specs.append(pl.BlockSpec((BV // 4, Fout), out_map))
        out_shape.append(jax.ShapeDtypeStruct((V // 4, Fout), jnp.float32))
    def body(*refs):
        if pool:
            (x0_r, g1_r, g2_r, w_r, b_r, gb_r, z_r, p_r, y_sc, s1, s2) = refs
        else:
            (x0_r, g1_r, g2_r, w_r, b_r, gb_r, z_r, y_sc, s1, s2) = refs
            p_r = None
        _combine_bn_body(x0_r, g1_r, g2_r, w_r, b_r, gb_r, z_r, p_r,
                         y_sc, s1, s2, Fin=Fin, nblocks=n, V=V, BV=BV,
                         pool=pool)

    res = pl.pallas_call(
        body,
        grid=(2 * n,),
        in_specs=in_specs,
        out_specs=out_specs if pool else out_specs[0],
        out_shape=out_shape if pool else out_shape[0],
        scratch_shapes=[pltpu.VMEM((V, Fout), jnp.float32),
                        pltpu.VMEM((1, Fout), jnp.float32),
                        pltpu.VMEM((1, Fout), jnp.float32)],
    )(x0, g1, g2, w_stack, b.reshape(1, Fout), gb)
    return res


def _bn_body(y_ref, st_ref, gb_ref, z_ref, *, V):
    m = st_ref[0:1, :] / V
    var = st_ref[1:2, :] / V - m * m
    inv = lax.rsqrt(var + _EPS)
    z_ref[...] = jnp.maximum(
        (y_ref[...] - m) * (inv * gb_ref[0:1, :]) + gb_ref[1:2, :], 0.0)


def _bn_pool_body(y_ref, st_ref, gb_ref, z_ref, p_ref, *, V):
    m = st_ref[0:1, :] / V
    var = st_ref[1:2, :] / V - m * m
    inv = lax.rsqrt(var + _EPS)
    z = jnp.maximum(
        (y_ref[...] - m) * (inv * gb_ref[0:1, :]) + gb_ref[1:2, :], 0.0)
    z_ref[...] = z
    bv, f = z.shape
    p_ref[...] = jnp.mean(z.reshape(bv // 4, 4, f), axis=1)


def _bn_relu(y, st, g, be, pool=False):
    V, F = y.shape
    BV = _row_block(V)
    gb = jnp.stack([g, be], axis=0)
    in_specs = [pl.BlockSpec((BV, F), lambda i: (i, 0)),
                pl.BlockSpec((2, F), lambda i: (0, 0)),
                pl.BlockSpec((2, F), lambda i: (0, 0))]
    if not pool:
        return pl.pallas_call(
            functools.partial(_bn_body, V=V),
            grid=(V // BV,),
            in_specs=in_specs,
            out_specs=pl.BlockSpec((BV, F), lambda i: (i, 0)),
            out_shape=jax.ShapeDtypeStruct((V, F), jnp.float32),
        )(y, st, gb)
    return pl.pallas_call(
        functools.partial(_bn_pool_body, V=V),
        grid=(V // BV,),
        in_specs=in_specs,
        out_specs=[pl.BlockSpec((BV, F), lambda i: (i, 0)),
                   pl.BlockSpec((BV // 4, F), lambda i: (i, 0))],
        out_shape=[jax.ShapeDtypeStruct((V, F), jnp.float32),
                   jax.ShapeDtypeStruct((V // 4, F), jnp.float32)],
    )(y, st, gb)


def _pool_body(x_ref, p_ref):
    p_ref[...] = jnp.mean(x_ref[...], axis=1)


def _pool(x):
    V, F = x.shape
    Vp = V // 4
    BP = _row_block(Vp)
    x3 = x.reshape(Vp, 4, F)
    return pl.pallas_call(
        _pool_body,
        grid=(Vp // BP,),
        in_specs=[pl.BlockSpec((BP, 4, F), lambda i: (i, 0, 0))],
        out_specs=pl.BlockSpec((BP, F), lambda i: (i, 0)),
        out_shape=jax.ShapeDtypeStruct((Vp, F), jnp.float32),
    )(x3)


def _unpool_body(x_ref, u_ref):
    b, _, f = u_ref.shape
    u_ref[...] = jnp.broadcast_to(x_ref[...][:, None, :], (b, 4, f))


def _unpool(x):
    Vc, F = x.shape
    BP = _row_block(Vc)
    u = pl.pallas_call(
        _unpool_body,
        grid=(Vc // BP,),
        in_specs=[pl.BlockSpec((BP, F), lambda i: (i, 0))],
        out_specs=pl.BlockSpec((BP, 4, F), lambda i: (i, 0, 0)),
        out_shape=jax.ShapeDtypeStruct((Vc, 4, F), jnp.float32),
    )(x)
    return u.reshape(4 * Vc, F)


def _prep_weights(w):
    K, Fin, Fout = w.shape
    weff = jnp.transpose(w.reshape(Fin, K, Fout), (1, 0, 2))
    return weff[0] - weff[2], weff[1], 2.0 * weff[2]


def _spmm_chain(lap, x0):
    if x0.shape[0] <= _SMALL_V:
        return _spmm_pair(lap, x0)
    g1 = _spmm(lap, x0)
    return g1, _spmm(lap, g1)


def _cheb_fin(lap, x0, p, name, with_stats):
    wa, wb, wc = _prep_weights(p[name + '_w'])
    w_stack = jnp.concatenate([wa, wb, wc], axis=0)
    g1, g2 = _spmm_chain(lap, x0)
    return _cheb_combine(x0, g1, g2, w_stack, p[name + '_b'], with_stats)


def _fout_weights(p, name):
    wa, wb, wc = _prep_weights(p[name + '_w'])
    b = p[name + '_b']
    Fout = wa.shape[1]
    wcat = jnp.concatenate([wc, wb, wa], axis=1)
    bias3 = jnp.concatenate(
        [jnp.zeros((2 * Fout,), jnp.float32), b]).reshape(1, 3 * Fout)
    return wcat, bias3


def _cheb_fout_x(lap, x0, p, name):
    wcat, bias3 = _fout_weights(p, name)
    q, a2, a3 = _mm3_x(x0, wcat, bias3)
    if q.shape[0] <= _SMALL_V:
        return _spmm_pair_add(lap, q, a2, a3)[1]
    t = _spmm(lap, q, addend=a2)
    return _spmm(lap, t, addend=a3)


def _cheb_fout_up(lap, zc, skip, p, name):
    wcat, bias3 = _fout_weights(p, name)
    Fu = zc.shape[1]
    q, a2, a3 = _mm3_up(zc, skip, wcat[:Fu], wcat[Fu:], bias3)
    if q.shape[0] <= _SMALL_V:
        return _spmm_pair_add(lap, q, a2, a3)[1]
    t = _spmm(lap, q, addend=a2)
    return _spmm(lap, t, addend=a3)


def _cbn_fin(lap, x0, p, name, pool=False):
    wa, wb, wc = _prep_weights(p[name + '_w'])
    w_stack = jnp.concatenate([wa, wb, wc], axis=0)
    g1, g2 = _spmm_chain(lap, x0)
    return _combine_bn(x0, g1, g2, w_stack, p[name + '_b'],
                       p[name + '_g'], p[name + '_be'], pool)


def kernel(x, params, laps):
    p = params
    x0 = x[0]

    x5a = _cbn_fin(laps[5], x0, p, 'e5a')
    x5, x5p = _cbn_fin(laps[5], x5a, p, 'e5b', pool=True)
    x4, x4p = _cbn_fin(laps[4], x5p, p, 'e4', pool=True)
    x3, x3p = _cbn_fin(laps[3], x4p, p, 'e3', pool=True)
    x2, x2p = _cbn_fin(laps[2], x3p, p, 'e2', pool=True)
    _, x1p = _cbn_fin(laps[1], x2p, p, 'e1', pool=True)
    xb = _cheb_fin(laps[0], x1p, p, 'e0', False)[0]

    z1 = _cbn_fin(laps[1], _unpool(xb), p, 'd1a')
    d1 = _cheb_fout_x(laps[1], z1, p, 'd1b')
    d2 = _stats_bn(_cheb_fout_up(laps[2], d1, x2, p, 'd2'),
                   p['d2_g'], p['d2_be'])
    d3 = _stats_bn(_cheb_fout_up(laps[3], d2, x3, p, 'd3'),
                   p['d3_g'], p['d3_be'])
    d4 = _stats_bn(_cheb_fout_up(laps[4], d3, x4, p, 'd4'),
                   p['d4_g'], p['d4_be'])
    d5 = _stats_bn(_cheb_fout_up(laps[5], d4, x5, p, 'd5'),
                   p['d5_g'], p['d5_be'])
    out = _cheb_fin(laps[5], d5, p, 'df', False)[0]
    return out.reshape(1, out.shape[0], 1)
